# baseline (speedup 1.0000x reference)
import jax
import jax.numpy as jnp
from jax.experimental import pallas as pl

B, T, DIM = 4, 16384, 3
C_DIM, HIDDEN, N_BLOCKS = 128, 128, 5
RESO, PADDING, SCALE = 128, 0.1, 8.0


def _normalize_coordinate(p, padding=PADDING):
    xy = p[:, :, jnp.array([0, 2])]
    xy = xy / (1.0 + padding + 1e-3) + 0.5
    return jnp.clip(xy, 0.0, 1.0 - 1e-3)


def _coordinate2index(xy, reso):
    g = (xy * reso).astype(jnp.int32)
    return g[:, :, 0] + reso * g[:, :, 1]


def _resnet_block(x, W0, b0, W1, b1, Ws):
    net = jax.nn.relu(x) @ W0 + b0
    dx = jax.nn.relu(net) @ W1 + b1
    return x @ Ws + dx


def _pool_local_max(c, idx, num_seg):
    def one(cb, ib):
        seg = jax.ops.segment_max(cb, ib, num_segments=num_seg)
        return seg[ib]
    return jax.vmap(one)(c, idx)


def _scatter_mean(c, idx, num_seg):
    def one(cb, ib):
        s = jax.ops.segment_sum(cb, ib, num_segments=num_seg)
        cnt = jax.ops.segment_sum(jnp.ones((cb.shape[0],), cb.dtype), ib, num_segments=num_seg)
        return s / jnp.maximum(cnt, 1.0)[:, None]
    return jax.vmap(one)(c, idx)


def _copy_kernel(x_ref, o_ref):
    o_ref[...] = x_ref[...]


def kernel(p, fc_pos_W, fc_pos_b, blocks_fc0_W, blocks_fc0_b, blocks_fc1_W, blocks_fc1_b, blocks_sc_W, fc_c_W, fc_c_b):
    xy = _normalize_coordinate(p / SCALE)
    idx = _coordinate2index(xy, RESO)
    net = p @ fc_pos_W + fc_pos_b
    net = _resnet_block(net, blocks_fc0_W[0], blocks_fc0_b[0], blocks_fc1_W[0], blocks_fc1_b[0], blocks_sc_W[0])
    for i in range(1, N_BLOCKS):
        pooled = _pool_local_max(net, idx, RESO * RESO)
        meanp = jnp.broadcast_to(jnp.mean(net, axis=-2, keepdims=True), net.shape)
        net = jnp.concatenate([net, pooled, meanp], axis=2)
        net = _resnet_block(net, blocks_fc0_W[i], blocks_fc0_b[i], blocks_fc1_W[i], blocks_fc1_b[i], blocks_sc_W[i])
    c = net @ fc_c_W + fc_c_b
    fea = _scatter_mean(c, idx, RESO * RESO)
    fea = jnp.transpose(fea, (0, 2, 1)).reshape(p.shape[0], C_DIM, RESO, RESO)
    fea = pl.pallas_call(
        _copy_kernel,
        out_shape=jax.ShapeDtypeStruct(fea.shape, fea.dtype),
        grid=(B,),
        in_specs=[pl.BlockSpec((1, C_DIM, RESO, RESO), lambda b: (b, 0, 0, 0))],
        out_specs=pl.BlockSpec((1, C_DIM, RESO, RESO), lambda b: (b, 0, 0, 0)),
    )(fea)
    return fea


# speedup vs baseline: 1.5082x; 1.5082x over previous
import functools

import jax
import jax.numpy as jnp
from jax import lax
from jax.experimental import pallas as pl
from jax.experimental.pallas import tpu as pltpu
from jax.experimental.pallas import tpu_sc as plsc

B, T, DIM = 4, 16384, 3
C_DIM, HIDDEN, N_BLOCKS = 128, 128, 5
RESO, PADDING, SCALE = 128, 0.1, 8.0
NSEG = RESO * RESO
H = HIDDEN
RC = 2048
NCHUNK = T // RC

NW = 32
BPW = NSEG // NW
L = 16
CH = 64

_NEG = -3.0e38


def _idx_body(px_ref, pz_ref, o_ref):
    scale = 1.0 / (SCALE * (1.0 + PADDING + 1e-3))
    x = jnp.clip(px_ref[0] * scale + 0.5, 0.0, 1.0 - 1e-3)
    z = jnp.clip(pz_ref[0] * scale + 0.5, 0.0, 1.0 - 1e-3)
    gx = (x * RESO).astype(jnp.int32)
    gz = (z * RESO).astype(jnp.int32)
    o_ref[0] = gx + RESO * gz


def _compute_idx(px, pz):
    return pl.pallas_call(
        _idx_body,
        out_shape=jax.ShapeDtypeStruct((B, RESO, RESO), jnp.int32),
        grid=(B,),
        in_specs=[
            pl.BlockSpec((1, RESO, RESO), lambda b: (b, 0, 0)),
            pl.BlockSpec((1, RESO, RESO), lambda b: (b, 0, 0)),
        ],
        out_specs=pl.BlockSpec((1, RESO, RESO), lambda b: (b, 0, 0)),
    )(px, pz)


def _dot(a, b):
    return jax.lax.dot_general(a, b, (((1,), (0,)), ((), ())),
                               preferred_element_type=jnp.float32,
                               precision=jax.lax.Precision.HIGHEST)


def _block0_body(p_ref, fpW_ref, fpb_ref, W0_ref, b0_ref, W1_ref, b1_ref,
                 Ws_ref, net_ref, sum_ref):
    c = pl.program_id(1)
    x = _dot(p_ref[0], fpW_ref[...]) + fpb_ref[...]
    h = _dot(jax.nn.relu(x), W0_ref[...]) + b0_ref[...]
    dx = _dot(jax.nn.relu(h), W1_ref[...]) + b1_ref[...]
    out = _dot(x, Ws_ref[...]) + dx
    net_ref[0] = out
    part = jnp.sum(out, axis=0, keepdims=True)

    @pl.when(c == 0)
    def _():
        sum_ref[0] = part

    @pl.when(c != 0)
    def _():
        sum_ref[0] = sum_ref[0] + part


def _run_block0(p, fpW, fpb, W0, b0, W1, b1, Ws):
    full = lambda shape: pl.BlockSpec(shape, lambda b, c: tuple(0 for _ in shape))
    return pl.pallas_call(
        _block0_body,
        out_shape=(
            jax.ShapeDtypeStruct((B, T, H), jnp.float32),
            jax.ShapeDtypeStruct((B, 1, H), jnp.float32),
        ),
        grid=(B, NCHUNK),
        in_specs=[
            pl.BlockSpec((1, RC, DIM), lambda b, c: (b, c, 0)),
            full((DIM, 3 * H)), full((1, 3 * H)),
            full((3 * H, H)), full((1, H)),
            full((H, H)), full((1, H)),
            full((3 * H, H)),
        ],
        out_specs=(
            pl.BlockSpec((1, RC, H), lambda b, c: (b, c, 0)),
            pl.BlockSpec((1, 1, H), lambda b, c: (b, 0, 0)),
        ),
    )(p, fpW, fpb, W0, b0, W1, b1, Ws)


def _blocki_body(last, net_in_ref, pool_ref, sum_in_ref,
                 W0n_ref, W0p_ref, W0m_ref, b0_ref, W1_ref, b1_ref,
                 Wsn_ref, Wsp_ref, Wsm_ref, fcc_ref, fcb_ref, *outs):
    c = pl.program_id(1)
    xn = net_in_ref[0]
    xp = pool_ref[0]
    xm = sum_in_ref[0] * (1.0 / T)
    h = (_dot(jax.nn.relu(xn), W0n_ref[...])
         + _dot(jax.nn.relu(xp), W0p_ref[...])
         + _dot(jax.nn.relu(xm), W0m_ref[...])
         + b0_ref[...])
    dx = _dot(jax.nn.relu(h), W1_ref[...]) + b1_ref[...]
    out = (_dot(xn, Wsn_ref[...]) + _dot(xp, Wsp_ref[...])
           + _dot(xm, Wsm_ref[...]) + dx)
    if last:
        (c_ref,) = outs
        c_ref[0] = _dot(out, fcc_ref[...]) + fcb_ref[...]
    else:
        net_ref, sum_ref = outs
        net_ref[0] = out
        part = jnp.sum(out, axis=0, keepdims=True)

        @pl.when(c == 0)
        def _():
            sum_ref[0] = part

        @pl.when(c != 0)
        def _():
            sum_ref[0] = sum_ref[0] + part


def _run_blocki(net, pooled, sum_in, W0n, W0p, W0m, b0, W1, b1,
                Wsn, Wsp, Wsm, fcc, fcb, last):
    full = lambda shape: pl.BlockSpec(shape, lambda b, c: tuple(0 for _ in shape))
    if last:
        out_shape = (jax.ShapeDtypeStruct((B, T, H), jnp.float32),)
        out_specs = (pl.BlockSpec((1, RC, H), lambda b, c: (b, c, 0)),)
    else:
        out_shape = (
            jax.ShapeDtypeStruct((B, T, H), jnp.float32),
            jax.ShapeDtypeStruct((B, 1, H), jnp.float32),
        )
        out_specs = (
            pl.BlockSpec((1, RC, H), lambda b, c: (b, c, 0)),
            pl.BlockSpec((1, 1, H), lambda b, c: (b, 0, 0)),
        )
    return pl.pallas_call(
        functools.partial(_blocki_body, last),
        out_shape=out_shape,
        grid=(B, NCHUNK),
        in_specs=[
            pl.BlockSpec((1, RC, H), lambda b, c: (b, c, 0)),
            pl.BlockSpec((1, RC, H), lambda b, c: (b, c, 0)),
            pl.BlockSpec((1, 1, H), lambda b, c: (b, 0, 0)),
            full((H, H)), full((H, H)), full((H, H)), full((1, H)),
            full((H, H)), full((1, H)),
            full((H, H)), full((H, H)), full((H, H)),
            full((H, H)), full((1, H)),
        ],
        out_specs=out_specs,
    )(net, pooled, sum_in, W0n, W0p, W0m, b0, W1, b1, Wsn, Wsp, Wsm, fcc, fcb)


def _mesh():
    return plsc.VectorSubcoreMesh(core_axis_name="c", subcore_axis_name="s")


def _worker_id():
    return lax.axis_index("s") * 2 + lax.axis_index("c")


def _sget(ref, i):
    return ref[pl.ds(i, L)][0]


def _select(idx_v, sel_pt, sel_bin, w, own_fn, bin_fn):
    lanes = lax.iota(jnp.int32, L)

    def body(t, cnt):
        v = idx_v[pl.ds(t * L, L)]
        own = own_fn(v, w)
        ones = own.astype(jnp.int32)
        pos = cnt + plsc.cumsum(ones) - 1
        plsc.store_scatter(sel_pt, [pos], t * L + lanes, mask=own)
        plsc.store_scatter(sel_bin, [pos], bin_fn(v, w), mask=own)
        return cnt + jnp.sum(ones)

    cnt = lax.fori_loop(0, T // L, body, jnp.int32(0))
    rup = ((cnt + CH - 1) // CH) * CH

    @pl.when(cnt > 0)
    def _():
        lp = jnp.full((L,), _sget(sel_pt, cnt - 1), jnp.int32)
        lb = jnp.full((L,), _sget(sel_bin, cnt - 1), jnp.int32)
        for m in range(0, CH, L):
            sel_pt[pl.ds(cnt + m, L)] = lp
            sel_bin[pl.ds(cnt + m, L)] = lb

    return cnt, rup


def _pool_sc(idx, net):
    @functools.partial(
        pl.kernel,
        mesh=_mesh(),
        compiler_params=pltpu.CompilerParams(needs_layout_passes=False),
        out_type=jax.ShapeDtypeStruct((B, T, H), jnp.float32),
        scratch_types=[
            pltpu.VMEM((T,), jnp.int32),
            pltpu.VMEM((T + 2 * CH,), jnp.int32),
            pltpu.VMEM((T + 2 * CH,), jnp.int32),
            pltpu.VMEM((BPW * H,), jnp.float32),
            pltpu.VMEM((CH, H), jnp.float32),
            pltpu.SemaphoreType.DMA,
            pltpu.SemaphoreType.DMA,
        ],
    )
    def kern(idx_hbm, net_hbm, out_hbm, idx_v, sel_pt, sel_bin, bins, gstage,
             sem, sem_w):
        w = _worker_id()
        neg = jnp.full((L,), _NEG, jnp.float32)

        def one_batch(b, _):
            pltpu.sync_copy(idx_hbm.at[b], idx_v)
            cnt, rup = _select(idx_v, sel_pt, sel_bin, w,
                               lambda v, w: (v & (NW - 1)) == w,
                               lambda v, w: v >> 5)
            nch = rup // CH

            def init_pt(q, _):
                base = _sget(sel_bin, q) * H
                for f in range(H // L):
                    bins[pl.ds(base + f * L, L)] = neg
                return 0

            lax.fori_loop(0, rup, init_pt, 0)

            def rmw_chunk(k, _):
                pltpu.async_copy(
                    net_hbm.at[b].at[sel_pt.at[pl.ds(k * CH, CH)]], gstage,
                    sem).wait()

                def one(j, _):
                    base = _sget(sel_bin, k * CH + j) * H
                    for f in range(H // L):
                        s = pl.ds(base + f * L, L)
                        bins[s] = jnp.maximum(bins[s],
                                              gstage[j, pl.ds(f * L, L)])
                    return 0

                return lax.fori_loop(0, CH, one, 0)

            lax.fori_loop(0, nch, rmw_chunk, 0)

            def gb_group(k, _):
                def fire(q, _):
                    base = _sget(sel_bin, q) * H
                    pt = _sget(sel_pt, q)
                    pltpu.async_copy(bins.at[pl.ds(base, H)],
                                     out_hbm.at[b, pt], sem_w)
                    return 0

                lax.fori_loop(k * CH, k * CH + CH, fire, 0)

                def drain(q, _):
                    pltpu.make_async_copy(out_hbm.at[b, 0],
                                          bins.at[pl.ds(0, H)], sem_w).wait()
                    return 0

                lax.fori_loop(0, CH, drain, 0)
                return 0

            lax.fori_loop(0, nch, gb_group, 0)
            return 0

        lax.fori_loop(0, B, one_batch, 0)

    return kern(idx, net)


def _scatter_mean_sc(idx, cfeat):
    @functools.partial(
        pl.kernel,
        mesh=_mesh(),
        compiler_params=pltpu.CompilerParams(needs_layout_passes=False),
        out_type=jax.ShapeDtypeStruct((B, C_DIM, RESO, RESO), jnp.float32),
        scratch_types=[
            pltpu.VMEM((T,), jnp.int32),
            pltpu.VMEM((T + 2 * CH,), jnp.int32),
            pltpu.VMEM((T + 2 * CH,), jnp.int32),
            pltpu.VMEM((H * BPW,), jnp.float32),
            pltpu.VMEM((BPW + L,), jnp.float32),
            pltpu.VMEM((BPW,), jnp.float32),
            pltpu.VMEM((CH, H), jnp.float32),
            pltpu.SemaphoreType.DMA,
            pltpu.SemaphoreType.DMA,
        ],
    )
    def kern(idx_hbm, c_hbm, out_hbm, idx_v, sel_pt, sel_bin, bins_t, cnt_v,
             inv_v, gstage, sem, sem_w):
        w = _worker_id()
        zeros = jnp.zeros((L,), jnp.float32)
        lanes = lax.iota(jnp.int32, L)

        def one_batch(b, _):
            def z0(i, _):
                bins_t[pl.ds(i * L, L)] = zeros
                return 0

            lax.fori_loop(0, H * BPW // L, z0, 0)

            def zc(i, _):
                cnt_v[pl.ds(i * L, L)] = zeros
                return 0

            lax.fori_loop(0, BPW // L, zc, 0)

            pltpu.sync_copy(idx_hbm.at[b], idx_v)
            cnt, rup = _select(idx_v, sel_pt, sel_bin, w,
                               lambda v, w: (v >> 9) == w,
                               lambda v, w: v & (BPW - 1))
            nch = rup // CH

            def rmw_chunk(k, _):
                pltpu.async_copy(
                    c_hbm.at[b].at[sel_pt.at[pl.ds(k * CH, CH)]], gstage,
                    sem).wait()

                def one(j, _):
                    q = k * CH + j

                    @pl.when(q < cnt)
                    def _():
                        bl = _sget(sel_bin, q)
                        blv = jnp.full((L,), bl, jnp.int32)
                        cv = plsc.load_gather(cnt_v, [blv]) + 1.0
                        plsc.store_scatter(cnt_v, [blv], cv, mask=lanes == 0)
                        for f in range(H // L):
                            iv = (lanes + f * L) * BPW + bl
                            cur = plsc.load_gather(bins_t, [iv])
                            plsc.store_scatter(
                                bins_t, [iv],
                                cur + gstage[j, pl.ds(f * L, L)])
                    return 0

                return lax.fori_loop(0, CH, one, 0)

            lax.fori_loop(0, nch, rmw_chunk, 0)

            def invc(i, _):
                cv = cnt_v[pl.ds(i * L, L)]
                inv_v[pl.ds(i * L, L)] = 1.0 / jnp.maximum(cv, 1.0)
                return 0

            lax.fori_loop(0, BPW // L, invc, 0)

            def drow(i, _):
                f, cg = i // (BPW // L), i % (BPW // L)
                s = pl.ds(f * BPW + cg * L, L)
                bins_t[s] = bins_t[s] * inv_v[pl.ds(cg * L, L)]
                return 0

            lax.fori_loop(0, H * BPW // L, drow, 0)

            def out_group(gk, _):
                def fire(q, _):
                    f = q >> 2
                    gy = w * (BPW // RESO) + (q & 3)
                    pltpu.async_copy(bins_t.at[pl.ds(q * RESO, RESO)],
                                     out_hbm.at[b, f, gy], sem_w)
                    return 0

                lax.fori_loop(gk * 64, gk * 64 + 64, fire, 0)

                def drain(q, _):
                    pltpu.make_async_copy(out_hbm.at[b, 0, 0],
                                          bins_t.at[pl.ds(0, RESO)],
                                          sem_w).wait()
                    return 0

                lax.fori_loop(0, 64, drain, 0)
                return 0

            lax.fori_loop(0, H * (BPW // RESO) // 64, out_group, 0)
            return 0

        lax.fori_loop(0, B, one_batch, 0)

    return kern(idx, cfeat)


def kernel(p, fc_pos_W, fc_pos_b, blocks_fc0_W, blocks_fc0_b, blocks_fc1_W,
           blocks_fc1_b, blocks_sc_W, fc_c_W, fc_c_b):
    px = p[:, :, 0].reshape(B, RESO, RESO)
    pz = p[:, :, 2].reshape(B, RESO, RESO)
    idx3 = _compute_idx(px, pz)
    idx = idx3.reshape(B, T)

    fpb = fc_pos_b.reshape(1, 3 * H)
    net, s = _run_block0(p, fc_pos_W, fpb,
                         blocks_fc0_W[0], blocks_fc0_b[0].reshape(1, H),
                         blocks_fc1_W[0], blocks_fc1_b[0].reshape(1, H),
                         blocks_sc_W[0])

    for i in range(1, N_BLOCKS):
        last = i == N_BLOCKS - 1
        pooled = _pool_sc(idx, net)
        W0 = blocks_fc0_W[i]
        Ws = blocks_sc_W[i]
        outs = _run_blocki(
            net, pooled, s,
            W0[:H], W0[H:2 * H], W0[2 * H:], blocks_fc0_b[i].reshape(1, H),
            blocks_fc1_W[i], blocks_fc1_b[i].reshape(1, H),
            Ws[:H], Ws[H:2 * H], Ws[2 * H:],
            fc_c_W, fc_c_b.reshape(1, H), last)
        if last:
            (cfeat,) = outs
        else:
            net, s = outs

    return _scatter_mean_sc(idx, cfeat)


# speedup vs baseline: 1.5517x; 1.0288x over previous
import functools

import jax
import jax.numpy as jnp
from jax import lax
from jax.experimental import pallas as pl
from jax.experimental.pallas import tpu as pltpu
from jax.experimental.pallas import tpu_sc as plsc

B, T, DIM = 4, 16384, 3
C_DIM, HIDDEN, N_BLOCKS = 128, 128, 5
RESO, PADDING, SCALE = 128, 0.1, 8.0
NSEG = RESO * RESO
H = HIDDEN
RC = 2048
NCHUNK = T // RC

NW = 32
BPW = NSEG // NW
L = 16
CH = 64

_NEG = -3.0e38


def _idx_body(px_ref, pz_ref, o_ref):
    scale = 1.0 / (SCALE * (1.0 + PADDING + 1e-3))
    x = jnp.clip(px_ref[0] * scale + 0.5, 0.0, 1.0 - 1e-3)
    z = jnp.clip(pz_ref[0] * scale + 0.5, 0.0, 1.0 - 1e-3)
    gx = (x * RESO).astype(jnp.int32)
    gz = (z * RESO).astype(jnp.int32)
    o_ref[0] = gx + RESO * gz


def _compute_idx(px, pz):
    return pl.pallas_call(
        _idx_body,
        out_shape=jax.ShapeDtypeStruct((B, RESO, RESO), jnp.int32),
        grid=(B,),
        in_specs=[
            pl.BlockSpec((1, RESO, RESO), lambda b: (b, 0, 0)),
            pl.BlockSpec((1, RESO, RESO), lambda b: (b, 0, 0)),
        ],
        out_specs=pl.BlockSpec((1, RESO, RESO), lambda b: (b, 0, 0)),
    )(px, pz)


def _dot(a, b):
    return jax.lax.dot_general(a, b, (((1,), (0,)), ((), ())),
                               preferred_element_type=jnp.float32,
                               precision=jax.lax.Precision.HIGHEST)


def _block0_body(p_ref, fpW_ref, fpb_ref, W0_ref, b0_ref, W1_ref, b1_ref,
                 Ws_ref, net_ref, sum_ref):
    c = pl.program_id(1)
    x = _dot(p_ref[0], fpW_ref[...]) + fpb_ref[...]
    h = _dot(jax.nn.relu(x), W0_ref[...]) + b0_ref[...]
    dx = _dot(jax.nn.relu(h), W1_ref[...]) + b1_ref[...]
    out = _dot(x, Ws_ref[...]) + dx
    net_ref[0] = out
    part = jnp.sum(out, axis=0, keepdims=True)

    @pl.when(c == 0)
    def _():
        sum_ref[0] = part

    @pl.when(c != 0)
    def _():
        sum_ref[0] = sum_ref[0] + part


def _run_block0(p, fpW, fpb, W0, b0, W1, b1, Ws):
    full = lambda shape: pl.BlockSpec(shape, lambda b, c: tuple(0 for _ in shape))
    return pl.pallas_call(
        _block0_body,
        out_shape=(
            jax.ShapeDtypeStruct((B, T, H), jnp.float32),
            jax.ShapeDtypeStruct((B, 1, H), jnp.float32),
        ),
        grid=(B, NCHUNK),
        in_specs=[
            pl.BlockSpec((1, RC, DIM), lambda b, c: (b, c, 0)),
            full((DIM, 3 * H)), full((1, 3 * H)),
            full((3 * H, H)), full((1, H)),
            full((H, H)), full((1, H)),
            full((3 * H, H)),
        ],
        out_specs=(
            pl.BlockSpec((1, RC, H), lambda b, c: (b, c, 0)),
            pl.BlockSpec((1, 1, H), lambda b, c: (b, 0, 0)),
        ),
    )(p, fpW, fpb, W0, b0, W1, b1, Ws)


def _blocki_body(last, net_in_ref, pool_ref, sum_in_ref,
                 W0n_ref, W0p_ref, W0m_ref, b0_ref, W1_ref, b1_ref,
                 Wsn_ref, Wsp_ref, Wsm_ref, fcc_ref, fcb_ref, *outs):
    c = pl.program_id(1)
    xn = net_in_ref[0]
    xp = pool_ref[0]
    xm = sum_in_ref[0] * (1.0 / T)
    h = (_dot(jax.nn.relu(xn), W0n_ref[...])
         + _dot(jax.nn.relu(xp), W0p_ref[...])
         + _dot(jax.nn.relu(xm), W0m_ref[...])
         + b0_ref[...])
    dx = _dot(jax.nn.relu(h), W1_ref[...]) + b1_ref[...]
    out = (_dot(xn, Wsn_ref[...]) + _dot(xp, Wsp_ref[...])
           + _dot(xm, Wsm_ref[...]) + dx)
    if last:
        (c_ref,) = outs
        c_ref[0] = _dot(out, fcc_ref[...]) + fcb_ref[...]
    else:
        net_ref, sum_ref = outs
        net_ref[0] = out
        part = jnp.sum(out, axis=0, keepdims=True)

        @pl.when(c == 0)
        def _():
            sum_ref[0] = part

        @pl.when(c != 0)
        def _():
            sum_ref[0] = sum_ref[0] + part


def _run_blocki(net, pooled, sum_in, W0n, W0p, W0m, b0, W1, b1,
                Wsn, Wsp, Wsm, fcc, fcb, last):
    full = lambda shape: pl.BlockSpec(shape, lambda b, c: tuple(0 for _ in shape))
    if last:
        out_shape = (jax.ShapeDtypeStruct((B, T, H), jnp.float32),)
        out_specs = (pl.BlockSpec((1, RC, H), lambda b, c: (b, c, 0)),)
    else:
        out_shape = (
            jax.ShapeDtypeStruct((B, T, H), jnp.float32),
            jax.ShapeDtypeStruct((B, 1, H), jnp.float32),
        )
        out_specs = (
            pl.BlockSpec((1, RC, H), lambda b, c: (b, c, 0)),
            pl.BlockSpec((1, 1, H), lambda b, c: (b, 0, 0)),
        )
    return pl.pallas_call(
        functools.partial(_blocki_body, last),
        out_shape=out_shape,
        grid=(B, NCHUNK),
        in_specs=[
            pl.BlockSpec((1, RC, H), lambda b, c: (b, c, 0)),
            pl.BlockSpec((1, RC, H), lambda b, c: (b, c, 0)),
            pl.BlockSpec((1, 1, H), lambda b, c: (b, 0, 0)),
            full((H, H)), full((H, H)), full((H, H)), full((1, H)),
            full((H, H)), full((1, H)),
            full((H, H)), full((H, H)), full((H, H)),
            full((H, H)), full((1, H)),
        ],
        out_specs=out_specs,
    )(net, pooled, sum_in, W0n, W0p, W0m, b0, W1, b1, Wsn, Wsp, Wsm, fcc, fcb)


def _mesh():
    return plsc.VectorSubcoreMesh(core_axis_name="c", subcore_axis_name="s")


def _worker_id():
    return lax.axis_index("s") * 2 + lax.axis_index("c")


def _sget(ref, i):
    return ref[pl.ds(i, L)][0]


def _select(idx_v, sel_pt, sel_bin, w, own_fn, bin_fn, pad_bin=None):
    lanes = lax.iota(jnp.int32, L)

    def body(t, cnt):
        v = idx_v[pl.ds(t * L, L)]
        own = own_fn(v, w)
        ones = own.astype(jnp.int32)
        pos = cnt + plsc.cumsum(ones) - 1
        plsc.store_scatter(sel_pt, [pos], t * L + lanes, mask=own)
        plsc.store_scatter(sel_bin, [pos], bin_fn(v, w), mask=own)
        return cnt + jnp.sum(ones)

    cnt = lax.fori_loop(0, T // L, body, jnp.int32(0), unroll=4)
    rup = ((cnt + CH - 1) // CH) * CH

    @pl.when(cnt > 0)
    def _():
        lp = jnp.full((L,), _sget(sel_pt, cnt - 1), jnp.int32)
        if pad_bin is None:
            lb = jnp.full((L,), _sget(sel_bin, cnt - 1), jnp.int32)
        else:
            lb = jnp.full((L,), pad_bin, jnp.int32)
        for m in range(0, CH, L):
            sel_pt[pl.ds(cnt + m, L)] = lp
            sel_bin[pl.ds(cnt + m, L)] = lb

    return cnt, rup


def _pool_sc(idx, net):
    @functools.partial(
        pl.kernel,
        mesh=_mesh(),
        compiler_params=pltpu.CompilerParams(needs_layout_passes=False),
        out_type=jax.ShapeDtypeStruct((B, T, H), jnp.float32),
        scratch_types=[
            pltpu.VMEM((T,), jnp.int32),
            pltpu.VMEM((T + 2 * CH,), jnp.int32),
            pltpu.VMEM((T + 2 * CH,), jnp.int32),
            pltpu.VMEM((BPW * H,), jnp.float32),
            pltpu.VMEM((CH, H), jnp.float32),
            pltpu.SemaphoreType.DMA,
            pltpu.SemaphoreType.DMA,
        ],
    )
    def kern(idx_hbm, net_hbm, out_hbm, idx_v, sel_pt, sel_bin, bins, gstage,
             sem, sem_w):
        w = _worker_id()
        neg = jnp.full((L,), _NEG, jnp.float32)

        def one_batch(b, _):
            pltpu.sync_copy(idx_hbm.at[b], idx_v)
            cnt, rup = _select(idx_v, sel_pt, sel_bin, w,
                               lambda v, w: (v & (NW - 1)) == w,
                               lambda v, w: v >> 5)
            nch = rup // CH

            def init_group(k, _):
                def init_pt(j, _):
                    base = _sget(sel_bin, k * CH + j) * H
                    for f in range(H // L):
                        bins[pl.ds(base + f * L, L)] = neg
                    return 0

                return lax.fori_loop(0, CH, init_pt, 0, unroll=4)

            lax.fori_loop(0, nch, init_group, 0)

            def rmw_chunk(k, _):
                pltpu.async_copy(
                    net_hbm.at[b].at[sel_pt.at[pl.ds(k * CH, CH)]], gstage,
                    sem).wait()

                def one(j, _):
                    base = _sget(sel_bin, k * CH + j) * H
                    for f in range(H // L):
                        s = pl.ds(base + f * L, L)
                        bins[s] = jnp.maximum(bins[s],
                                              gstage[j, pl.ds(f * L, L)])
                    return 0

                return lax.fori_loop(0, CH, one, 0, unroll=2)

            lax.fori_loop(0, nch, rmw_chunk, 0)

            def gb_group(k, _):
                def fire(j, _):
                    q = k * CH + j
                    base = _sget(sel_bin, q) * H
                    pt = _sget(sel_pt, q)
                    pltpu.async_copy(bins.at[pl.ds(base, H)],
                                     out_hbm.at[b, pt], sem_w)
                    return 0

                lax.fori_loop(0, CH, fire, 0, unroll=4)

                def drain(j, _):
                    pltpu.make_async_copy(out_hbm.at[b, 0],
                                          bins.at[pl.ds(0, H)], sem_w).wait()
                    return 0

                lax.fori_loop(0, CH, drain, 0, unroll=4)
                return 0

            lax.fori_loop(0, nch, gb_group, 0)
            return 0

        lax.fori_loop(0, B, one_batch, 0)

    return kern(idx, net)


def _scatter_mean_sc(idx, cfeat):
    @functools.partial(
        pl.kernel,
        mesh=_mesh(),
        compiler_params=pltpu.CompilerParams(needs_layout_passes=False),
        out_type=jax.ShapeDtypeStruct((B, C_DIM, RESO, RESO), jnp.float32),
        scratch_types=[
            pltpu.VMEM((T,), jnp.int32),
            pltpu.VMEM((T + 2 * CH,), jnp.int32),
            pltpu.VMEM((T + 2 * CH,), jnp.int32),
            pltpu.VMEM((H * BPW,), jnp.float32),
            pltpu.VMEM((BPW + 1 + L,), jnp.float32),
            pltpu.VMEM((BPW,), jnp.float32),
            pltpu.VMEM((CH, H), jnp.float32),
            pltpu.SemaphoreType.DMA,
            pltpu.SemaphoreType.DMA,
        ],
    )
    def kern(idx_hbm, c_hbm, out_hbm, idx_v, sel_pt, sel_bin, bins_t, cnt_v,
             inv_v, gstage, sem, sem_w):
        w = _worker_id()
        zeros = jnp.zeros((L,), jnp.float32)
        lanes = lax.iota(jnp.int32, L)
        S = BPW

        def one_batch(b, _):
            def z0(i, _):
                bins_t[pl.ds(i * L, L)] = zeros
                return 0

            lax.fori_loop(0, H * S // L, z0, 0, unroll=8)

            def zc(i, _):
                cnt_v[pl.ds(i * L, L)] = zeros
                return 0

            lax.fori_loop(0, S // L + 1, zc, 0, unroll=4)

            pltpu.sync_copy(idx_hbm.at[b], idx_v)
            cnt, rup = _select(idx_v, sel_pt, sel_bin, w,
                               lambda v, w: (v >> 9) == w,
                               lambda v, w: v & (BPW - 1))
            nch = rup // CH

            def rmw_chunk(k, _):
                pltpu.async_copy(
                    c_hbm.at[b].at[sel_pt.at[pl.ds(k * CH, CH)]], gstage,
                    sem).wait()

                def one(j, _):
                    q = k * CH + j
                    real = jnp.full((L,), q < cnt)
                    bl = _sget(sel_bin, q)
                    blv = jnp.full((L,), bl, jnp.int32)
                    cv = plsc.load_gather(cnt_v, [blv]) + 1.0
                    plsc.store_scatter(cnt_v, [blv], cv,
                                       mask=(lanes == 0) & real)
                    for f in range(H // L):
                        iv = (lanes + f * L) * S + bl
                        cur = plsc.load_gather(bins_t, [iv])
                        plsc.store_scatter(
                            bins_t, [iv],
                            cur + gstage[j, pl.ds(f * L, L)], mask=real)
                    return 0

                return lax.fori_loop(0, CH, one, 0, unroll=2)

            lax.fori_loop(0, nch, rmw_chunk, 0)

            def invc(i, _):
                cv = cnt_v[pl.ds(i * L, L)]
                inv_v[pl.ds(i * L, L)] = 1.0 / jnp.maximum(cv, 1.0)
                return 0

            lax.fori_loop(0, BPW // L, invc, 0, unroll=4)

            def drow(i, _):
                f, cg = i // (BPW // L), i % (BPW // L)
                s = pl.ds(f * S + cg * L, L)
                bins_t[s] = bins_t[s] * inv_v[pl.ds(cg * L, L)]
                return 0

            lax.fori_loop(0, H * BPW // L, drow, 0, unroll=8)

            def out_group(gk, _):
                def fire(j, _):
                    q = gk * 64 + j
                    f = q >> 2
                    g = q & 3
                    pltpu.async_copy(bins_t.at[pl.ds(f * S + g * RESO, RESO)],
                                     out_hbm.at[b, f, w * (BPW // RESO) + g],
                                     sem_w)
                    return 0

                lax.fori_loop(0, 64, fire, 0, unroll=4)

                def drain(j, _):
                    pltpu.make_async_copy(out_hbm.at[b, 0, 0],
                                          bins_t.at[pl.ds(0, RESO)],
                                          sem_w).wait()
                    return 0

                lax.fori_loop(0, 64, drain, 0, unroll=4)
                return 0

            lax.fori_loop(0, H * (BPW // RESO) // 64, out_group, 0)
            return 0

        lax.fori_loop(0, B, one_batch, 0)

    return kern(idx, cfeat)


def kernel(p, fc_pos_W, fc_pos_b, blocks_fc0_W, blocks_fc0_b, blocks_fc1_W,
           blocks_fc1_b, blocks_sc_W, fc_c_W, fc_c_b):
    px = p[:, :, 0].reshape(B, RESO, RESO)
    pz = p[:, :, 2].reshape(B, RESO, RESO)
    idx3 = _compute_idx(px, pz)
    idx = idx3.reshape(B, T)

    fpb = fc_pos_b.reshape(1, 3 * H)
    net, s = _run_block0(p, fc_pos_W, fpb,
                         blocks_fc0_W[0], blocks_fc0_b[0].reshape(1, H),
                         blocks_fc1_W[0], blocks_fc1_b[0].reshape(1, H),
                         blocks_sc_W[0])

    for i in range(1, N_BLOCKS):
        last = i == N_BLOCKS - 1
        pooled = _pool_sc(idx, net)
        W0 = blocks_fc0_W[i]
        Ws = blocks_sc_W[i]
        outs = _run_blocki(
            net, pooled, s,
            W0[:H], W0[H:2 * H], W0[2 * H:], blocks_fc0_b[i].reshape(1, H),
            blocks_fc1_W[i], blocks_fc1_b[i].reshape(1, H),
            Ws[:H], Ws[H:2 * H], Ws[2 * H:],
            fc_c_W, fc_c_b.reshape(1, H), last)
        if last:
            (cfeat,) = outs
        else:
            net, s = outs

    return _scatter_mean_sc(idx, cfeat)


# speedup vs baseline: 1.5524x; 1.0004x over previous
import functools

import jax
import jax.numpy as jnp
from jax import lax
from jax.experimental import pallas as pl
from jax.experimental.pallas import tpu as pltpu
from jax.experimental.pallas import tpu_sc as plsc

B, T, DIM = 4, 16384, 3
C_DIM, HIDDEN, N_BLOCKS = 128, 128, 5
RESO, PADDING, SCALE = 128, 0.1, 8.0
NSEG = RESO * RESO
H = HIDDEN
RC = 2048
NCHUNK = T // RC

NW = 32
BPW = NSEG // NW
L = 16
CH = 64

_NEG = -3.0e38


def _idx_body(px_ref, pz_ref, o_ref):
    scale = 1.0 / (SCALE * (1.0 + PADDING + 1e-3))
    x = jnp.clip(px_ref[0] * scale + 0.5, 0.0, 1.0 - 1e-3)
    z = jnp.clip(pz_ref[0] * scale + 0.5, 0.0, 1.0 - 1e-3)
    gx = (x * RESO).astype(jnp.int32)
    gz = (z * RESO).astype(jnp.int32)
    o_ref[0] = gx + RESO * gz


def _compute_idx(px, pz):
    return pl.pallas_call(
        _idx_body,
        out_shape=jax.ShapeDtypeStruct((B, RESO, RESO), jnp.int32),
        grid=(B,),
        in_specs=[
            pl.BlockSpec((1, RESO, RESO), lambda b: (b, 0, 0)),
            pl.BlockSpec((1, RESO, RESO), lambda b: (b, 0, 0)),
        ],
        out_specs=pl.BlockSpec((1, RESO, RESO), lambda b: (b, 0, 0)),
    )(px, pz)


def _dot(a, b):
    return jax.lax.dot_general(a, b, (((1,), (0,)), ((), ())),
                               preferred_element_type=jnp.float32,
                               precision=jax.lax.Precision.HIGHEST)


def _block0_body(p_ref, fpW_ref, fpb_ref, W0_ref, b0_ref, W1_ref, b1_ref,
                 Ws_ref, net_ref, sum_ref):
    c = pl.program_id(1)
    x = _dot(p_ref[0], fpW_ref[...]) + fpb_ref[...]
    h = _dot(jax.nn.relu(x), W0_ref[...]) + b0_ref[...]
    dx = _dot(jax.nn.relu(h), W1_ref[...]) + b1_ref[...]
    out = _dot(x, Ws_ref[...]) + dx
    net_ref[0] = out
    part = jnp.sum(out, axis=0, keepdims=True)

    @pl.when(c == 0)
    def _():
        sum_ref[0] = part

    @pl.when(c != 0)
    def _():
        sum_ref[0] = sum_ref[0] + part


def _run_block0(p, fpW, fpb, W0, b0, W1, b1, Ws):
    full = lambda shape: pl.BlockSpec(shape, lambda b, c: tuple(0 for _ in shape))
    return pl.pallas_call(
        _block0_body,
        out_shape=(
            jax.ShapeDtypeStruct((B, T, H), jnp.float32),
            jax.ShapeDtypeStruct((B, 1, H), jnp.float32),
        ),
        grid=(B, NCHUNK),
        in_specs=[
            pl.BlockSpec((1, RC, DIM), lambda b, c: (b, c, 0)),
            full((DIM, 3 * H)), full((1, 3 * H)),
            full((3 * H, H)), full((1, H)),
            full((H, H)), full((1, H)),
            full((3 * H, H)),
        ],
        out_specs=(
            pl.BlockSpec((1, RC, H), lambda b, c: (b, c, 0)),
            pl.BlockSpec((1, 1, H), lambda b, c: (b, 0, 0)),
        ),
    )(p, fpW, fpb, W0, b0, W1, b1, Ws)


def _blocki_body(last, net_in_ref, pool_ref, sum_in_ref,
                 W0n_ref, W0p_ref, W0m_ref, b0_ref, W1_ref, b1_ref,
                 Wsn_ref, Wsp_ref, Wsm_ref, fcc_ref, fcb_ref, *outs):
    c = pl.program_id(1)
    xn = net_in_ref[0]
    xp = pool_ref[0]
    xm = sum_in_ref[0] * (1.0 / T)
    h = (_dot(jax.nn.relu(xn), W0n_ref[...])
         + _dot(jax.nn.relu(xp), W0p_ref[...])
         + _dot(jax.nn.relu(xm), W0m_ref[...])
         + b0_ref[...])
    dx = _dot(jax.nn.relu(h), W1_ref[...]) + b1_ref[...]
    out = (_dot(xn, Wsn_ref[...]) + _dot(xp, Wsp_ref[...])
           + _dot(xm, Wsm_ref[...]) + dx)
    if last:
        (c_ref,) = outs
        c_ref[0] = _dot(out, fcc_ref[...]) + fcb_ref[...]
    else:
        net_ref, sum_ref = outs
        net_ref[0] = out
        part = jnp.sum(out, axis=0, keepdims=True)

        @pl.when(c == 0)
        def _():
            sum_ref[0] = part

        @pl.when(c != 0)
        def _():
            sum_ref[0] = sum_ref[0] + part


def _run_blocki(net, pooled, sum_in, W0n, W0p, W0m, b0, W1, b1,
                Wsn, Wsp, Wsm, fcc, fcb, last):
    full = lambda shape: pl.BlockSpec(shape, lambda b, c: tuple(0 for _ in shape))
    if last:
        out_shape = (jax.ShapeDtypeStruct((B, T, H), jnp.float32),)
        out_specs = (pl.BlockSpec((1, RC, H), lambda b, c: (b, c, 0)),)
    else:
        out_shape = (
            jax.ShapeDtypeStruct((B, T, H), jnp.float32),
            jax.ShapeDtypeStruct((B, 1, H), jnp.float32),
        )
        out_specs = (
            pl.BlockSpec((1, RC, H), lambda b, c: (b, c, 0)),
            pl.BlockSpec((1, 1, H), lambda b, c: (b, 0, 0)),
        )
    return pl.pallas_call(
        functools.partial(_blocki_body, last),
        out_shape=out_shape,
        grid=(B, NCHUNK),
        in_specs=[
            pl.BlockSpec((1, RC, H), lambda b, c: (b, c, 0)),
            pl.BlockSpec((1, RC, H), lambda b, c: (b, c, 0)),
            pl.BlockSpec((1, 1, H), lambda b, c: (b, 0, 0)),
            full((H, H)), full((H, H)), full((H, H)), full((1, H)),
            full((H, H)), full((1, H)),
            full((H, H)), full((H, H)), full((H, H)),
            full((H, H)), full((1, H)),
        ],
        out_specs=out_specs,
    )(net, pooled, sum_in, W0n, W0p, W0m, b0, W1, b1, Wsn, Wsp, Wsm, fcc, fcb)


def _mesh():
    return plsc.VectorSubcoreMesh(core_axis_name="c", subcore_axis_name="s")


def _worker_id():
    return lax.axis_index("s") * 2 + lax.axis_index("c")


def _sget(ref, i):
    return ref[pl.ds(i, L)][0]


def _select(idx_v, sel_pt, sel_bin, w, own_fn, bin_fn, pad_bin=None):
    lanes = lax.iota(jnp.int32, L)

    def body(t, cnt):
        v = idx_v[pl.ds(t * L, L)]
        own = own_fn(v, w)
        ones = own.astype(jnp.int32)
        pos = cnt + plsc.cumsum(ones) - 1
        plsc.store_scatter(sel_pt, [pos], t * L + lanes, mask=own)
        plsc.store_scatter(sel_bin, [pos], bin_fn(v, w), mask=own)
        return cnt + jnp.sum(ones)

    cnt = lax.fori_loop(0, T // L, body, jnp.int32(0), unroll=4)
    rup = ((cnt + CH - 1) // CH) * CH

    @pl.when(cnt > 0)
    def _():
        lp = jnp.full((L,), _sget(sel_pt, cnt - 1), jnp.int32)
        if pad_bin is None:
            lb = jnp.full((L,), _sget(sel_bin, cnt - 1), jnp.int32)
        else:
            lb = jnp.full((L,), pad_bin, jnp.int32)
        for m in range(0, CH, L):
            sel_pt[pl.ds(cnt + m, L)] = lp
            sel_bin[pl.ds(cnt + m, L)] = lb

    return cnt, rup


def _pool_sc(idx, net):
    @functools.partial(
        pl.kernel,
        mesh=_mesh(),
        compiler_params=pltpu.CompilerParams(needs_layout_passes=False),
        out_type=jax.ShapeDtypeStruct((B, T, H), jnp.float32),
        scratch_types=[
            pltpu.VMEM((T,), jnp.int32),
            pltpu.VMEM((T + 2 * CH,), jnp.int32),
            pltpu.VMEM((T + 2 * CH,), jnp.int32),
            pltpu.VMEM((BPW * H,), jnp.float32),
            pltpu.VMEM((CH, H), jnp.float32),
            pltpu.SemaphoreType.DMA,
            pltpu.SemaphoreType.DMA,
        ],
    )
    def kern(idx_hbm, net_hbm, out_hbm, idx_v, sel_pt, sel_bin, bins, gstage,
             sem, sem_w):
        w = _worker_id()
        neg = jnp.full((L,), _NEG, jnp.float32)

        def one_batch(b, _):
            pltpu.sync_copy(idx_hbm.at[b], idx_v)
            with jax.named_scope("psel"):
                cnt, rup = _select(idx_v, sel_pt, sel_bin, w,
                                   lambda v, w: (v & (NW - 1)) == w,
                                   lambda v, w: v >> 5)
            nch = rup // CH

            def init_group(k, _):
                def init_pt(j, _):
                    base = _sget(sel_bin, k * CH + j) * H
                    for f in range(H // L):
                        bins[pl.ds(base + f * L, L)] = neg
                    return 0

                return lax.fori_loop(0, CH, init_pt, 0, unroll=4)

            with jax.named_scope("pinit"):
                lax.fori_loop(0, nch, init_group, 0)

            def rmw_chunk(k, _):
                pltpu.async_copy(
                    net_hbm.at[b].at[sel_pt.at[pl.ds(k * CH, CH)]], gstage,
                    sem).wait()

                def one(j, _):
                    base = _sget(sel_bin, k * CH + j) * H
                    for f in range(H // L):
                        s = pl.ds(base + f * L, L)
                        bins[s] = jnp.maximum(bins[s],
                                              gstage[j, pl.ds(f * L, L)])
                    return 0

                return lax.fori_loop(0, CH, one, 0, unroll=2)

            with jax.named_scope("prmw"):
                lax.fori_loop(0, nch, rmw_chunk, 0)

            def gb_group(k, _):
                def fire(j, _):
                    q = k * CH + j
                    base = _sget(sel_bin, q) * H
                    pt = _sget(sel_pt, q)
                    pltpu.async_copy(bins.at[pl.ds(base, H)],
                                     out_hbm.at[b, pt], sem_w)
                    return 0

                lax.fori_loop(0, CH, fire, 0, unroll=4)

                def drain(j, _):
                    pltpu.make_async_copy(out_hbm.at[b, 0],
                                          bins.at[pl.ds(0, H)], sem_w).wait()
                    return 0

                lax.fori_loop(0, CH, drain, 0, unroll=4)
                return 0

            with jax.named_scope("pgb"):
                lax.fori_loop(0, nch, gb_group, 0)
            return 0

        lax.fori_loop(0, B, one_batch, 0)

    return kern(idx, net)


def _scatter_mean_sc(idx, cfeat):
    @functools.partial(
        pl.kernel,
        mesh=_mesh(),
        compiler_params=pltpu.CompilerParams(needs_layout_passes=False),
        out_type=jax.ShapeDtypeStruct((B, C_DIM, RESO, RESO), jnp.float32),
        scratch_types=[
            pltpu.VMEM((T,), jnp.int32),
            pltpu.VMEM((T + 2 * CH,), jnp.int32),
            pltpu.VMEM((T + 2 * CH,), jnp.int32),
            pltpu.VMEM((H * BPW,), jnp.float32),
            pltpu.VMEM((BPW + 1 + L,), jnp.float32),
            pltpu.VMEM((BPW,), jnp.float32),
            pltpu.VMEM((CH, H), jnp.float32),
            pltpu.SemaphoreType.DMA,
            pltpu.SemaphoreType.DMA,
        ],
    )
    def kern(idx_hbm, c_hbm, out_hbm, idx_v, sel_pt, sel_bin, bins_t, cnt_v,
             inv_v, gstage, sem, sem_w):
        w = _worker_id()
        zeros = jnp.zeros((L,), jnp.float32)
        lanes = lax.iota(jnp.int32, L)
        S = BPW

        def one_batch(b, _):
            def z0(i, _):
                bins_t[pl.ds(i * L, L)] = zeros
                return 0

            with jax.named_scope("mz0"):
                lax.fori_loop(0, H * S // L, z0, 0, unroll=8)

            def zc(i, _):
                cnt_v[pl.ds(i * L, L)] = zeros
                return 0

            lax.fori_loop(0, S // L + 1, zc, 0, unroll=4)

            pltpu.sync_copy(idx_hbm.at[b], idx_v)
            with jax.named_scope("msel"):
                cnt, rup = _select(idx_v, sel_pt, sel_bin, w,
                                   lambda v, w: (v >> 9) == w,
                                   lambda v, w: v & (BPW - 1))
            nch = rup // CH

            def rmw_chunk(k, _):
                pltpu.async_copy(
                    c_hbm.at[b].at[sel_pt.at[pl.ds(k * CH, CH)]], gstage,
                    sem).wait()

                def one(j, _):
                    q = k * CH + j
                    real = jnp.full((L,), q < cnt)
                    bl = _sget(sel_bin, q)
                    blv = jnp.full((L,), bl, jnp.int32)
                    cv = plsc.load_gather(cnt_v, [blv]) + 1.0
                    plsc.store_scatter(cnt_v, [blv], cv,
                                       mask=(lanes == 0) & real)
                    for f in range(H // L):
                        iv = (lanes + f * L) * S + bl
                        cur = plsc.load_gather(bins_t, [iv])
                        plsc.store_scatter(
                            bins_t, [iv],
                            cur + gstage[j, pl.ds(f * L, L)], mask=real)
                    return 0

                return lax.fori_loop(0, CH, one, 0, unroll=2)

            with jax.named_scope("mrmw"):
                lax.fori_loop(0, nch, rmw_chunk, 0)

            def invc(i, _):
                cv = cnt_v[pl.ds(i * L, L)]
                inv_v[pl.ds(i * L, L)] = 1.0 / jnp.maximum(cv, 1.0)
                return 0

            lax.fori_loop(0, BPW // L, invc, 0, unroll=4)

            def drow(i, _):
                f, cg = i // (BPW // L), i % (BPW // L)
                s = pl.ds(f * S + cg * L, L)
                bins_t[s] = bins_t[s] * inv_v[pl.ds(cg * L, L)]
                return 0

            with jax.named_scope("mdiv"):
                lax.fori_loop(0, H * BPW // L, drow, 0, unroll=8)

            def out_group(gk, _):
                def fire(j, _):
                    q = gk * 64 + j
                    f = q >> 2
                    g = q & 3
                    pltpu.async_copy(bins_t.at[pl.ds(f * S + g * RESO, RESO)],
                                     out_hbm.at[b, f, w * (BPW // RESO) + g],
                                     sem_w)
                    return 0

                lax.fori_loop(0, 64, fire, 0, unroll=4)

                def drain(j, _):
                    pltpu.make_async_copy(out_hbm.at[b, 0, 0],
                                          bins_t.at[pl.ds(0, RESO)],
                                          sem_w).wait()
                    return 0

                lax.fori_loop(0, 64, drain, 0, unroll=4)
                return 0

            with jax.named_scope("mout"):
                lax.fori_loop(0, H * (BPW // RESO) // 64, out_group, 0)
            return 0

        lax.fori_loop(0, B, one_batch, 0)

    return kern(idx, cfeat)


def kernel(p, fc_pos_W, fc_pos_b, blocks_fc0_W, blocks_fc0_b, blocks_fc1_W,
           blocks_fc1_b, blocks_sc_W, fc_c_W, fc_c_b):
    px = p[:, :, 0].reshape(B, RESO, RESO)
    pz = p[:, :, 2].reshape(B, RESO, RESO)
    idx3 = _compute_idx(px, pz)
    idx = idx3.reshape(B, T)

    fpb = fc_pos_b.reshape(1, 3 * H)
    net, s = _run_block0(p, fc_pos_W, fpb,
                         blocks_fc0_W[0], blocks_fc0_b[0].reshape(1, H),
                         blocks_fc1_W[0], blocks_fc1_b[0].reshape(1, H),
                         blocks_sc_W[0])

    for i in range(1, N_BLOCKS):
        last = i == N_BLOCKS - 1
        pooled = _pool_sc(idx, net)
        W0 = blocks_fc0_W[i]
        Ws = blocks_sc_W[i]
        outs = _run_blocki(
            net, pooled, s,
            W0[:H], W0[H:2 * H], W0[2 * H:], blocks_fc0_b[i].reshape(1, H),
            blocks_fc1_W[i], blocks_fc1_b[i].reshape(1, H),
            Ws[:H], Ws[H:2 * H], Ws[2 * H:],
            fc_c_W, fc_c_b.reshape(1, H), last)
        if last:
            (cfeat,) = outs
        else:
            net, s = outs

    return _scatter_mean_sc(idx, cfeat)


# speedup vs baseline: 3.1805x; 2.0487x over previous
import functools

import jax
import jax.numpy as jnp
from jax import lax
from jax.experimental import pallas as pl
from jax.experimental.pallas import tpu as pltpu
from jax.experimental.pallas import tpu_sc as plsc

B, T, DIM = 4, 16384, 3
C_DIM, HIDDEN, N_BLOCKS = 128, 128, 5
RESO, PADDING, SCALE = 128, 0.1, 8.0
NSEG = RESO * RESO
H = HIDDEN
RC = 2048
NCHUNK = T // RC

NW = 32
BPW = NSEG // NW
L = 16
CH = 64

_NEG = -3.0e38


def _idx_body(px_ref, pz_ref, o_ref):
    scale = 1.0 / (SCALE * (1.0 + PADDING + 1e-3))
    x = jnp.clip(px_ref[0] * scale + 0.5, 0.0, 1.0 - 1e-3)
    z = jnp.clip(pz_ref[0] * scale + 0.5, 0.0, 1.0 - 1e-3)
    gx = (x * RESO).astype(jnp.int32)
    gz = (z * RESO).astype(jnp.int32)
    o_ref[0] = gx + RESO * gz


def _compute_idx(px, pz):
    return pl.pallas_call(
        _idx_body,
        out_shape=jax.ShapeDtypeStruct((B, RESO, RESO), jnp.int32),
        grid=(B,),
        in_specs=[
            pl.BlockSpec((1, RESO, RESO), lambda b: (b, 0, 0)),
            pl.BlockSpec((1, RESO, RESO), lambda b: (b, 0, 0)),
        ],
        out_specs=pl.BlockSpec((1, RESO, RESO), lambda b: (b, 0, 0)),
    )(px, pz)


def _dot(a, b):
    return jax.lax.dot_general(a, b, (((1,), (0,)), ((), ())),
                               preferred_element_type=jnp.float32)


def _block0_body(p_ref, fpW_ref, fpb_ref, W0_ref, b0_ref, W1_ref, b1_ref,
                 Ws_ref, net_ref, sum_ref):
    c = pl.program_id(1)
    x = _dot(p_ref[0], fpW_ref[...]) + fpb_ref[...]
    h = _dot(jax.nn.relu(x), W0_ref[...]) + b0_ref[...]
    dx = _dot(jax.nn.relu(h), W1_ref[...]) + b1_ref[...]
    out = _dot(x, Ws_ref[...]) + dx
    net_ref[0] = out
    part = jnp.sum(out, axis=0, keepdims=True)

    @pl.when(c == 0)
    def _():
        sum_ref[0] = part

    @pl.when(c != 0)
    def _():
        sum_ref[0] = sum_ref[0] + part


def _run_block0(p, fpW, fpb, W0, b0, W1, b1, Ws):
    full = lambda shape: pl.BlockSpec(shape, lambda b, c: tuple(0 for _ in shape))
    return pl.pallas_call(
        _block0_body,
        out_shape=(
            jax.ShapeDtypeStruct((B, T, H), jnp.float32),
            jax.ShapeDtypeStruct((B, 1, H), jnp.float32),
        ),
        grid=(B, NCHUNK),
        in_specs=[
            pl.BlockSpec((1, RC, DIM), lambda b, c: (b, c, 0)),
            full((DIM, 3 * H)), full((1, 3 * H)),
            full((3 * H, H)), full((1, H)),
            full((H, H)), full((1, H)),
            full((3 * H, H)),
        ],
        out_specs=(
            pl.BlockSpec((1, RC, H), lambda b, c: (b, c, 0)),
            pl.BlockSpec((1, 1, H), lambda b, c: (b, 0, 0)),
        ),
    )(p, fpW, fpb, W0, b0, W1, b1, Ws)


def _blocki_body(last, net_in_ref, pool_ref, sum_in_ref,
                 W0np_ref, W0m_ref, b0_ref, W1_ref, b1_ref,
                 Wsnp_ref, Wsm_ref, fcc_ref, fcb_ref, *outs):
    c = pl.program_id(1)
    xn = net_in_ref[0]
    xp = pool_ref[0]
    xm = sum_in_ref[0] * (1.0 / T)
    xcat = jnp.concatenate([xn, xp], axis=1)
    h = (_dot(jax.nn.relu(xcat), W0np_ref[...])
         + _dot(jax.nn.relu(xm), W0m_ref[...])
         + b0_ref[...])
    dx = _dot(jax.nn.relu(h), W1_ref[...]) + b1_ref[...]
    out = (_dot(xcat, Wsnp_ref[...])
           + _dot(xm, Wsm_ref[...]) + dx)
    if last:
        (c_ref,) = outs
        c_ref[0] = _dot(out, fcc_ref[...]) + fcb_ref[...]
    else:
        net_ref, sum_ref = outs
        net_ref[0] = out
        part = jnp.sum(out, axis=0, keepdims=True)

        @pl.when(c == 0)
        def _():
            sum_ref[0] = part

        @pl.when(c != 0)
        def _():
            sum_ref[0] = sum_ref[0] + part


def _run_blocki(net, pooled, sum_in, W0np, W0m, b0, W1, b1,
                Wsnp, Wsm, fcc, fcb, last):
    full = lambda shape: pl.BlockSpec(shape, lambda b, c: tuple(0 for _ in shape))
    if last:
        out_shape = (jax.ShapeDtypeStruct((B, T, H), jnp.float32),)
        out_specs = (pl.BlockSpec((1, RC, H), lambda b, c: (b, c, 0)),)
    else:
        out_shape = (
            jax.ShapeDtypeStruct((B, T, H), jnp.float32),
            jax.ShapeDtypeStruct((B, 1, H), jnp.float32),
        )
        out_specs = (
            pl.BlockSpec((1, RC, H), lambda b, c: (b, c, 0)),
            pl.BlockSpec((1, 1, H), lambda b, c: (b, 0, 0)),
        )
    return pl.pallas_call(
        functools.partial(_blocki_body, last),
        out_shape=out_shape,
        grid=(B, NCHUNK),
        in_specs=[
            pl.BlockSpec((1, RC, H), lambda b, c: (b, c, 0)),
            pl.BlockSpec((1, RC, H), lambda b, c: (b, c, 0)),
            pl.BlockSpec((1, 1, H), lambda b, c: (b, 0, 0)),
            full((2 * H, H)), full((H, H)), full((1, H)),
            full((H, H)), full((1, H)),
            full((2 * H, H)), full((H, H)),
            full((H, H)), full((1, H)),
        ],
        out_specs=out_specs,
    )(net, pooled, sum_in, W0np, W0m, b0, W1, b1, Wsnp, Wsm, fcc, fcb)


def _mesh():
    return plsc.VectorSubcoreMesh(core_axis_name="c", subcore_axis_name="s")


def _worker_id():
    return lax.axis_index("s") * 2 + lax.axis_index("c")


def _sget(ref, i):
    return ref[pl.ds(i, L)][0]


def _select(idx_v, sel_pt, sel_bin, w, own_fn, bin_fn, pad_bin=None):
    lanes = lax.iota(jnp.int32, L)

    def body(t, cnt):
        v = idx_v[pl.ds(t * L, L)]
        own = own_fn(v, w)
        ones = own.astype(jnp.int32)
        pos = cnt + plsc.cumsum(ones) - 1
        plsc.store_scatter(sel_pt, [pos], t * L + lanes, mask=own)
        plsc.store_scatter(sel_bin, [pos], bin_fn(v, w), mask=own)
        return cnt + jnp.sum(ones)

    cnt = lax.fori_loop(0, T // L, body, jnp.int32(0), unroll=4)
    rup = ((cnt + CH - 1) // CH) * CH

    @pl.when(cnt > 0)
    def _():
        lp = jnp.full((L,), _sget(sel_pt, cnt - 1), jnp.int32)
        if pad_bin is None:
            lb = jnp.full((L,), _sget(sel_bin, cnt - 1), jnp.int32)
        else:
            lb = jnp.full((L,), pad_bin, jnp.int32)
        for m in range(0, CH, L):
            sel_pt[pl.ds(cnt + m, L)] = lp
            sel_bin[pl.ds(cnt + m, L)] = lb

    return cnt, rup


def _pool_sc(idx, net):
    @functools.partial(
        pl.kernel,
        mesh=_mesh(),
        compiler_params=pltpu.CompilerParams(needs_layout_passes=False),
        out_type=jax.ShapeDtypeStruct((B, T, H), jnp.float32),
        scratch_types=[
            pltpu.VMEM((T,), jnp.int32),
            pltpu.VMEM((T + 2 * CH,), jnp.int32),
            pltpu.VMEM((T + 2 * CH,), jnp.int32),
            pltpu.VMEM((BPW * H,), jnp.float32),
            pltpu.VMEM((CH, H), jnp.float32),
            pltpu.SemaphoreType.DMA,
            pltpu.SemaphoreType.DMA,
        ],
    )
    def kern(idx_hbm, net_hbm, out_hbm, idx_v, sel_pt, sel_bin, bins, gstage,
             sem, sem_w):
        w = _worker_id()
        neg = jnp.full((L,), _NEG, jnp.float32)

        def one_batch(b, _):
            pltpu.sync_copy(idx_hbm.at[b], idx_v)
            with jax.named_scope("psel"):
                cnt, rup = _select(idx_v, sel_pt, sel_bin, w,
                                   lambda v, w: (v & (NW - 1)) == w,
                                   lambda v, w: v >> 5)
            nch = rup // CH

            def init_group(k, _):
                def init_pt(j, _):
                    base = _sget(sel_bin, k * CH + j) * H
                    for f in range(H // L):
                        bins[pl.ds(base + f * L, L)] = neg
                    return 0

                return lax.fori_loop(0, CH, init_pt, 0, unroll=4)

            with jax.named_scope("pinit"):
                lax.fori_loop(0, nch, init_group, 0)

            def rmw_chunk(k, _):
                pltpu.async_copy(
                    net_hbm.at[b].at[sel_pt.at[pl.ds(k * CH, CH)]], gstage,
                    sem).wait()

                def one(j, _):
                    base = _sget(sel_bin, k * CH + j) * H
                    for f in range(H // L):
                        s = pl.ds(base + f * L, L)
                        bins[s] = jnp.maximum(bins[s],
                                              gstage[j, pl.ds(f * L, L)])
                    return 0

                return lax.fori_loop(0, CH, one, 0, unroll=2)

            with jax.named_scope("prmw"):
                lax.fori_loop(0, nch, rmw_chunk, 0)

            def gb_group(k, _):
                def fire(j, _):
                    q = k * CH + j
                    base = _sget(sel_bin, q) * H
                    pt = _sget(sel_pt, q)
                    pltpu.async_copy(bins.at[pl.ds(base, H)],
                                     out_hbm.at[b, pt], sem_w)
                    return 0

                lax.fori_loop(0, CH, fire, 0, unroll=4)

                def drain(j, _):
                    pltpu.make_async_copy(out_hbm.at[b, 0],
                                          bins.at[pl.ds(0, H)], sem_w).wait()
                    return 0

                lax.fori_loop(0, CH, drain, 0, unroll=4)
                return 0

            with jax.named_scope("pgb"):
                lax.fori_loop(0, nch, gb_group, 0)
            return 0

        lax.fori_loop(0, B, one_batch, 0)

    return kern(idx, net)


def _scatter_mean_sc(idx, cfeat):
    @functools.partial(
        pl.kernel,
        mesh=_mesh(),
        compiler_params=pltpu.CompilerParams(needs_layout_passes=False),
        out_type=jax.ShapeDtypeStruct((B, C_DIM, RESO, RESO), jnp.float32),
        scratch_types=[
            pltpu.VMEM((T,), jnp.int32),
            pltpu.VMEM((T + 2 * CH,), jnp.int32),
            pltpu.VMEM((T + 2 * CH,), jnp.int32),
            pltpu.VMEM((BPW * H,), jnp.float32),
            pltpu.VMEM((BPW + L,), jnp.float32),
            pltpu.VMEM((BPW,), jnp.float32),
            pltpu.VMEM((CH, H), jnp.float32),
            pltpu.VMEM((8, BPW), jnp.float32),
            pltpu.SemaphoreType.DMA,
            pltpu.SemaphoreType.DMA,
        ],
    )
    def kern(idx_hbm, c_hbm, out_hbm, idx_v, sel_pt, sel_bin, bins, cnt_v,
             inv_v, gstage, ring, sem, sem_w):
        w = _worker_id()
        zeros = jnp.zeros((L,), jnp.float32)
        lanes = lax.iota(jnp.int32, L)
        NG = BPW // RESO

        def one_batch(b, _):
            def z0(i, _):
                bins[pl.ds(i * L, L)] = zeros
                return 0

            with jax.named_scope("mz0"):
                lax.fori_loop(0, BPW * H // L, z0, 0, unroll=8)

            def zc(i, _):
                cnt_v[pl.ds(i * L, L)] = zeros
                return 0

            lax.fori_loop(0, BPW // L, zc, 0, unroll=4)

            pltpu.sync_copy(idx_hbm.at[b], idx_v)
            with jax.named_scope("msel"):
                cnt, rup = _select(
                    idx_v, sel_pt, sel_bin, w,
                    lambda v, w: ((v >> 7) & (NW - 1)) == w,
                    lambda v, w: ((v >> 12) << 7) | (v & (RESO - 1)))
            nch = rup // CH

            def rmw_chunk(k, _):
                pltpu.async_copy(
                    c_hbm.at[b].at[sel_pt.at[pl.ds(k * CH, CH)]], gstage,
                    sem).wait()

                def one(j, _):
                    q = k * CH + j

                    @pl.when(q < cnt)
                    def _():
                        bl = _sget(sel_bin, q)
                        blv = jnp.full((L,), bl, jnp.int32)
                        cv = plsc.load_gather(cnt_v, [blv]) + 1.0
                        plsc.store_scatter(cnt_v, [blv], cv)
                        base = bl * H
                        for f in range(H // L):
                            s = pl.ds(base + f * L, L)
                            bins[s] = bins[s] + gstage[j, pl.ds(f * L, L)]
                    return 0

                return lax.fori_loop(0, CH, one, 0, unroll=2)

            with jax.named_scope("mrmw"):
                lax.fori_loop(0, nch, rmw_chunk, 0)

            def invc(i, _):
                cv = cnt_v[pl.ds(i * L, L)]
                inv_v[pl.ds(i * L, L)] = 1.0 / jnp.maximum(cv, 1.0)
                return 0

            lax.fori_loop(0, BPW // L, invc, 0, unroll=4)

            def orow(f, _):
                r = f & 7

                @pl.when(f >= 8)
                def _():
                    def drain(j, _):
                        pltpu.make_async_copy(out_hbm.at[b, 0, 0],
                                              ring.at[0, pl.ds(0, RESO)],
                                              sem_w).wait()
                        return 0
                    lax.fori_loop(0, NG, drain, 0, unroll=4)

                def gcol(i, _):
                    iv = (i * L + lanes) * H + f
                    vals = plsc.load_gather(bins, [iv]) * inv_v[pl.ds(i * L, L)]
                    ring[r, pl.ds(i * L, L)] = vals
                    return 0

                lax.fori_loop(0, BPW // L, gcol, 0, unroll=4)

                for g in range(NG):
                    pltpu.async_copy(ring.at[r, pl.ds(g * RESO, RESO)],
                                     out_hbm.at[b, f, w + NW * g], sem_w)
                return 0

            with jax.named_scope("mout"):
                lax.fori_loop(0, H, orow, 0)

                def draintail(j, _):
                    pltpu.make_async_copy(out_hbm.at[b, 0, 0],
                                          ring.at[0, pl.ds(0, RESO)],
                                          sem_w).wait()
                    return 0

                lax.fori_loop(0, 8 * NG, draintail, 0, unroll=4)
            return 0

        lax.fori_loop(0, B, one_batch, 0)

    return kern(idx, cfeat)


def kernel(p, fc_pos_W, fc_pos_b, blocks_fc0_W, blocks_fc0_b, blocks_fc1_W,
           blocks_fc1_b, blocks_sc_W, fc_c_W, fc_c_b):
    px = p[:, :, 0].reshape(B, RESO, RESO)
    pz = p[:, :, 2].reshape(B, RESO, RESO)
    idx3 = _compute_idx(px, pz)
    idx = idx3.reshape(B, T)

    fpb = fc_pos_b.reshape(1, 3 * H)
    net, s = _run_block0(p, fc_pos_W, fpb,
                         blocks_fc0_W[0], blocks_fc0_b[0].reshape(1, H),
                         blocks_fc1_W[0], blocks_fc1_b[0].reshape(1, H),
                         blocks_sc_W[0])

    for i in range(1, N_BLOCKS):
        last = i == N_BLOCKS - 1
        pooled = _pool_sc(idx, net)
        W0 = blocks_fc0_W[i]
        Ws = blocks_sc_W[i]
        outs = _run_blocki(
            net, pooled, s,
            W0[:2 * H], W0[2 * H:], blocks_fc0_b[i].reshape(1, H),
            blocks_fc1_W[i], blocks_fc1_b[i].reshape(1, H),
            Ws[:2 * H], Ws[2 * H:],
            fc_c_W, fc_c_b.reshape(1, H), last)
        if last:
            (cfeat,) = outs
        else:
            net, s = outs

    return _scatter_mean_sc(idx, cfeat)


# speedup vs baseline: 3.4021x; 1.0697x over previous
import functools

import jax
import jax.numpy as jnp
from jax import lax
from jax.experimental import pallas as pl
from jax.experimental.pallas import tpu as pltpu
from jax.experimental.pallas import tpu_sc as plsc

B, T, DIM = 4, 16384, 3
C_DIM, HIDDEN, N_BLOCKS = 128, 128, 5
RESO, PADDING, SCALE = 128, 0.1, 8.0
NSEG = RESO * RESO
H = HIDDEN
RC = 2048
NCHUNK = T // RC

NW = 32
BPW = NSEG // NW
L = 16
CH = 64

_NEG = -3.0e38


def _idx_body(px_ref, pz_ref, o_ref):
    scale = 1.0 / (SCALE * (1.0 + PADDING + 1e-3))
    x = jnp.clip(px_ref[0] * scale + 0.5, 0.0, 1.0 - 1e-3)
    z = jnp.clip(pz_ref[0] * scale + 0.5, 0.0, 1.0 - 1e-3)
    gx = (x * RESO).astype(jnp.int32)
    gz = (z * RESO).astype(jnp.int32)
    o_ref[0] = gx + RESO * gz


def _compute_idx(px, pz):
    return pl.pallas_call(
        _idx_body,
        out_shape=jax.ShapeDtypeStruct((B, RESO, RESO), jnp.int32),
        grid=(B,),
        in_specs=[
            pl.BlockSpec((1, RESO, RESO), lambda b: (b, 0, 0)),
            pl.BlockSpec((1, RESO, RESO), lambda b: (b, 0, 0)),
        ],
        out_specs=pl.BlockSpec((1, RESO, RESO), lambda b: (b, 0, 0)),
    )(px, pz)


def _dot(a, b):
    return jax.lax.dot_general(a, b, (((1,), (0,)), ((), ())),
                               preferred_element_type=jnp.float32)


def _block0_body(p_ref, fpW_ref, fpb_ref, W0_ref, b0_ref, W1_ref, b1_ref,
                 Ws_ref, net_ref, sum_ref):
    c = pl.program_id(1)
    x = _dot(p_ref[0], fpW_ref[...]) + fpb_ref[...]
    h = _dot(jax.nn.relu(x), W0_ref[...]) + b0_ref[...]
    dx = _dot(jax.nn.relu(h), W1_ref[...]) + b1_ref[...]
    out = _dot(x, Ws_ref[...]) + dx
    net_ref[0] = out
    part = jnp.sum(out, axis=0, keepdims=True)

    @pl.when(c == 0)
    def _():
        sum_ref[0] = part

    @pl.when(c != 0)
    def _():
        sum_ref[0] = sum_ref[0] + part


def _run_block0(p, fpW, fpb, W0, b0, W1, b1, Ws):
    full = lambda shape: pl.BlockSpec(shape, lambda b, c: tuple(0 for _ in shape))
    return pl.pallas_call(
        _block0_body,
        out_shape=(
            jax.ShapeDtypeStruct((B, T, H), jnp.float32),
            jax.ShapeDtypeStruct((B, 1, H), jnp.float32),
        ),
        grid=(B, NCHUNK),
        in_specs=[
            pl.BlockSpec((1, RC, DIM), lambda b, c: (b, c, 0)),
            full((DIM, 3 * H)), full((1, 3 * H)),
            full((3 * H, H)), full((1, H)),
            full((H, H)), full((1, H)),
            full((3 * H, H)),
        ],
        out_specs=(
            pl.BlockSpec((1, RC, H), lambda b, c: (b, c, 0)),
            pl.BlockSpec((1, 1, H), lambda b, c: (b, 0, 0)),
        ),
    )(p, fpW, fpb, W0, b0, W1, b1, Ws)


def _blocki_body(last, net_in_ref, pool_ref, sum_in_ref,
                 W0np_ref, W0m_ref, b0_ref, W1_ref, b1_ref,
                 Wsnp_ref, Wsm_ref, fcc_ref, fcb_ref, *outs):
    c = pl.program_id(1)
    xn = net_in_ref[0]
    xp = pool_ref[0]
    xm = sum_in_ref[0] * (1.0 / T)
    xcat = jnp.concatenate([xn, xp], axis=1)
    h = (_dot(jax.nn.relu(xcat), W0np_ref[...])
         + _dot(jax.nn.relu(xm), W0m_ref[...])
         + b0_ref[...])
    dx = _dot(jax.nn.relu(h), W1_ref[...]) + b1_ref[...]
    out = (_dot(xcat, Wsnp_ref[...])
           + _dot(xm, Wsm_ref[...]) + dx)
    if last:
        (c_ref,) = outs
        c_ref[0] = _dot(out, fcc_ref[...]) + fcb_ref[...]
    else:
        net_ref, sum_ref = outs
        net_ref[0] = out
        part = jnp.sum(out, axis=0, keepdims=True)

        @pl.when(c == 0)
        def _():
            sum_ref[0] = part

        @pl.when(c != 0)
        def _():
            sum_ref[0] = sum_ref[0] + part


def _run_blocki(net, pooled, sum_in, W0np, W0m, b0, W1, b1,
                Wsnp, Wsm, fcc, fcb, last):
    full = lambda shape: pl.BlockSpec(shape, lambda b, c: tuple(0 for _ in shape))
    if last:
        out_shape = (jax.ShapeDtypeStruct((B, T, H), jnp.float32),)
        out_specs = (pl.BlockSpec((1, RC, H), lambda b, c: (b, c, 0)),)
    else:
        out_shape = (
            jax.ShapeDtypeStruct((B, T, H), jnp.float32),
            jax.ShapeDtypeStruct((B, 1, H), jnp.float32),
        )
        out_specs = (
            pl.BlockSpec((1, RC, H), lambda b, c: (b, c, 0)),
            pl.BlockSpec((1, 1, H), lambda b, c: (b, 0, 0)),
        )
    return pl.pallas_call(
        functools.partial(_blocki_body, last),
        out_shape=out_shape,
        grid=(B, NCHUNK),
        in_specs=[
            pl.BlockSpec((1, RC, H), lambda b, c: (b, c, 0)),
            pl.BlockSpec((1, RC, H), lambda b, c: (b, c, 0)),
            pl.BlockSpec((1, 1, H), lambda b, c: (b, 0, 0)),
            full((2 * H, H)), full((H, H)), full((1, H)),
            full((H, H)), full((1, H)),
            full((2 * H, H)), full((H, H)),
            full((H, H)), full((1, H)),
        ],
        out_specs=out_specs,
    )(net, pooled, sum_in, W0np, W0m, b0, W1, b1, Wsnp, Wsm, fcc, fcb)


def _mesh():
    return plsc.VectorSubcoreMesh(core_axis_name="c", subcore_axis_name="s")


def _worker_id():
    return lax.axis_index("s") * 2 + lax.axis_index("c")


def _sget(ref, i):
    return ref[pl.ds(i, L)][0]


def _select(idx_v, sel_pt, sel_bin, w, own_fn, bin_fn, pad_bin=None):
    lanes = lax.iota(jnp.int32, L)

    def body(t, cnt):
        v = idx_v[pl.ds(t * L, L)]
        own = own_fn(v, w)
        ones = own.astype(jnp.int32)
        pos = cnt + plsc.cumsum(ones) - 1
        plsc.store_scatter(sel_pt, [pos], t * L + lanes, mask=own)
        plsc.store_scatter(sel_bin, [pos], bin_fn(v, w), mask=own)
        return cnt + jnp.sum(ones)

    cnt = lax.fori_loop(0, T // L, body, jnp.int32(0), unroll=4)
    rup = ((cnt + CH - 1) // CH) * CH

    @pl.when(cnt > 0)
    def _():
        lp = jnp.full((L,), _sget(sel_pt, cnt - 1), jnp.int32)
        if pad_bin is None:
            lb = jnp.full((L,), _sget(sel_bin, cnt - 1), jnp.int32)
        else:
            lb = jnp.full((L,), pad_bin, jnp.int32)
        for m in range(0, CH, L):
            sel_pt[pl.ds(cnt + m, L)] = lp
            sel_bin[pl.ds(cnt + m, L)] = lb

    return cnt, rup


def _pool_sc(idx, net):
    @functools.partial(
        pl.kernel,
        mesh=_mesh(),
        compiler_params=pltpu.CompilerParams(needs_layout_passes=False),
        out_type=jax.ShapeDtypeStruct((B, T, H), jnp.float32),
        scratch_types=[
            pltpu.VMEM((T,), jnp.int32),
            pltpu.VMEM((T + 2 * CH,), jnp.int32),
            pltpu.VMEM((T + 2 * CH,), jnp.int32),
            pltpu.VMEM((BPW * H,), jnp.float32),
            pltpu.VMEM((CH, H), jnp.float32),
            pltpu.SemaphoreType.DMA,
            pltpu.SemaphoreType.DMA,
        ],
    )
    def kern(idx_hbm, net_hbm, out_hbm, idx_v, sel_pt, sel_bin, bins, gstage,
             sem, sem_w):
        w = _worker_id()
        neg = jnp.full((L,), _NEG, jnp.float32)

        def one_batch(b, _):
            pltpu.sync_copy(idx_hbm.at[b], idx_v)
            with jax.named_scope("psel"):
                cnt, rup = _select(idx_v, sel_pt, sel_bin, w,
                                   lambda v, w: (v & (NW - 1)) == w,
                                   lambda v, w: v >> 5)
            nch = rup // CH

            def init_group(k, _):
                def init_pt(j, _):
                    base = _sget(sel_bin, k * CH + j) * H
                    for f in range(H // L):
                        bins[pl.ds(base + f * L, L)] = neg
                    return 0

                return lax.fori_loop(0, CH, init_pt, 0, unroll=4)

            with jax.named_scope("pinit"):
                lax.fori_loop(0, nch, init_group, 0)

            def rmw_chunk(k, _):
                pltpu.async_copy(
                    net_hbm.at[b].at[sel_pt.at[pl.ds(k * CH, CH)]], gstage,
                    sem).wait()

                def one(j, _):
                    base = _sget(sel_bin, k * CH + j) * H
                    for f in range(H // L):
                        s = pl.ds(base + f * L, L)
                        bins[s] = jnp.maximum(bins[s],
                                              gstage[j, pl.ds(f * L, L)])
                    return 0

                return lax.fori_loop(0, CH, one, 0, unroll=4)

            with jax.named_scope("prmw"):
                lax.fori_loop(0, nch, rmw_chunk, 0)

            def gb_group(k, _):
                def fire(j, _):
                    q = k * CH + j
                    base = _sget(sel_bin, q) * H
                    pt = _sget(sel_pt, q)
                    pltpu.async_copy(bins.at[pl.ds(base, H)],
                                     out_hbm.at[b, pt], sem_w)
                    return 0

                lax.fori_loop(0, CH, fire, 0, unroll=4)

                def drain(j, _):
                    pltpu.make_async_copy(out_hbm.at[b, 0],
                                          bins.at[pl.ds(0, H)], sem_w).wait()
                    return 0

                lax.fori_loop(0, CH, drain, 0, unroll=4)
                return 0

            with jax.named_scope("pgb"):
                lax.fori_loop(0, nch, gb_group, 0)
            return 0

        lax.fori_loop(0, B, one_batch, 0)

    return kern(idx, net)


def _scatter_mean_sc(idx, cfeat):
    @functools.partial(
        pl.kernel,
        mesh=_mesh(),
        compiler_params=pltpu.CompilerParams(needs_layout_passes=False),
        out_type=jax.ShapeDtypeStruct((B, C_DIM, RESO, RESO), jnp.float32),
        scratch_types=[
            pltpu.VMEM((T,), jnp.int32),
            pltpu.VMEM((T + 2 * CH,), jnp.int32),
            pltpu.VMEM((T + 2 * CH,), jnp.int32),
            pltpu.VMEM((BPW * (H + 1),), jnp.float32),
            pltpu.VMEM((BPW + L,), jnp.float32),
            pltpu.VMEM((BPW,), jnp.float32),
            pltpu.VMEM((CH, H), jnp.float32),
            pltpu.VMEM((8, BPW), jnp.float32),
            pltpu.SemaphoreType.DMA,
            pltpu.SemaphoreType.DMA,
        ],
    )
    def kern(idx_hbm, c_hbm, out_hbm, idx_v, sel_pt, sel_bin, bins, cnt_v,
             inv_v, gstage, ring, sem, sem_w):
        w = _worker_id()
        zeros = jnp.zeros((L,), jnp.float32)
        lanes = lax.iota(jnp.int32, L)
        NG = BPW // RESO
        SM = H + 1

        def one_batch(b, _):
            def z0(i, _):
                bins[pl.ds(i * L, L)] = zeros
                return 0

            with jax.named_scope("mz0"):
                lax.fori_loop(0, BPW * SM // L, z0, 0, unroll=8)

            def zc(i, _):
                cnt_v[pl.ds(i * L, L)] = zeros
                return 0

            lax.fori_loop(0, BPW // L, zc, 0, unroll=4)

            pltpu.sync_copy(idx_hbm.at[b], idx_v)
            with jax.named_scope("msel"):
                cnt, rup = _select(
                    idx_v, sel_pt, sel_bin, w,
                    lambda v, w: ((v >> 7) & (NW - 1)) == w,
                    lambda v, w: ((v >> 12) << 7) | (v & (RESO - 1)))
            nch = rup // CH

            def rmw_chunk(k, _):
                pltpu.async_copy(
                    c_hbm.at[b].at[sel_pt.at[pl.ds(k * CH, CH)]], gstage,
                    sem).wait()

                def one(j, _):
                    q = k * CH + j

                    @pl.when(q < cnt)
                    def _():
                        bl = _sget(sel_bin, q)
                        blv = jnp.full((L,), bl, jnp.int32)
                        cv = plsc.load_gather(cnt_v, [blv]) + 1.0
                        plsc.store_scatter(cnt_v, [blv], cv)
                        base = bl * SM
                        for f in range(H // L):
                            s = pl.ds(base + f * L, L)
                            bins[s] = bins[s] + gstage[j, pl.ds(f * L, L)]
                    return 0

                return lax.fori_loop(0, CH, one, 0, unroll=4)

            with jax.named_scope("mrmw"):
                lax.fori_loop(0, nch, rmw_chunk, 0)

            def invc(i, _):
                cv = cnt_v[pl.ds(i * L, L)]
                inv_v[pl.ds(i * L, L)] = 1.0 / jnp.maximum(cv, 1.0)
                return 0

            lax.fori_loop(0, BPW // L, invc, 0, unroll=4)

            def orow(f, _):
                r = f & 7

                @pl.when(f >= 8)
                def _():
                    def drain(j, _):
                        pltpu.make_async_copy(out_hbm.at[b, 0, 0],
                                              ring.at[0, pl.ds(0, RESO)],
                                              sem_w).wait()
                        return 0
                    lax.fori_loop(0, NG, drain, 0, unroll=4)

                def gcol(i, _):
                    iv = (i * L + lanes) * SM + f
                    vals = plsc.load_gather(bins, [iv]) * inv_v[pl.ds(i * L, L)]
                    ring[r, pl.ds(i * L, L)] = vals
                    return 0

                lax.fori_loop(0, BPW // L, gcol, 0, unroll=4)

                for g in range(NG):
                    pltpu.async_copy(ring.at[r, pl.ds(g * RESO, RESO)],
                                     out_hbm.at[b, f, w + NW * g], sem_w)
                return 0

            with jax.named_scope("mout"):
                lax.fori_loop(0, H, orow, 0)

                def draintail(j, _):
                    pltpu.make_async_copy(out_hbm.at[b, 0, 0],
                                          ring.at[0, pl.ds(0, RESO)],
                                          sem_w).wait()
                    return 0

                lax.fori_loop(0, 8 * NG, draintail, 0, unroll=4)
            return 0

        lax.fori_loop(0, B, one_batch, 0)

    return kern(idx, cfeat)


def kernel(p, fc_pos_W, fc_pos_b, blocks_fc0_W, blocks_fc0_b, blocks_fc1_W,
           blocks_fc1_b, blocks_sc_W, fc_c_W, fc_c_b):
    px = p[:, :, 0].reshape(B, RESO, RESO)
    pz = p[:, :, 2].reshape(B, RESO, RESO)
    idx3 = _compute_idx(px, pz)
    idx = idx3.reshape(B, T)

    fpb = fc_pos_b.reshape(1, 3 * H)
    net, s = _run_block0(p, fc_pos_W, fpb,
                         blocks_fc0_W[0], blocks_fc0_b[0].reshape(1, H),
                         blocks_fc1_W[0], blocks_fc1_b[0].reshape(1, H),
                         blocks_sc_W[0])

    for i in range(1, N_BLOCKS):
        last = i == N_BLOCKS - 1
        pooled = _pool_sc(idx, net)
        W0 = blocks_fc0_W[i]
        Ws = blocks_sc_W[i]
        outs = _run_blocki(
            net, pooled, s,
            W0[:2 * H], W0[2 * H:], blocks_fc0_b[i].reshape(1, H),
            blocks_fc1_W[i], blocks_fc1_b[i].reshape(1, H),
            Ws[:2 * H], Ws[2 * H:],
            fc_c_W, fc_c_b.reshape(1, H), last)
        if last:
            (cfeat,) = outs
        else:
            net, s = outs

    return _scatter_mean_sc(idx, cfeat)


# speedup vs baseline: 3.8067x; 1.1189x over previous
import functools

import jax
import jax.numpy as jnp
from jax import lax
from jax.experimental import pallas as pl
from jax.experimental.pallas import tpu as pltpu
from jax.experimental.pallas import tpu_sc as plsc

B, T, DIM = 4, 16384, 3
C_DIM, HIDDEN, N_BLOCKS = 128, 128, 5
RESO, PADDING, SCALE = 128, 0.1, 8.0
NSEG = RESO * RESO
H = HIDDEN
RC = 2048
NCHUNK = T // RC

NW = 32
BPW = NSEG // NW
L = 16
CH = 48

_NEG = -3.0e38


def _idx_body(px_ref, pz_ref, o_ref):
    scale = 1.0 / (SCALE * (1.0 + PADDING + 1e-3))
    x = jnp.clip(px_ref[0] * scale + 0.5, 0.0, 1.0 - 1e-3)
    z = jnp.clip(pz_ref[0] * scale + 0.5, 0.0, 1.0 - 1e-3)
    gx = (x * RESO).astype(jnp.int32)
    gz = (z * RESO).astype(jnp.int32)
    o_ref[0] = gx + RESO * gz


def _compute_idx(px, pz):
    return pl.pallas_call(
        _idx_body,
        out_shape=jax.ShapeDtypeStruct((B, RESO, RESO), jnp.int32),
        grid=(B,),
        in_specs=[
            pl.BlockSpec((1, RESO, RESO), lambda b: (b, 0, 0)),
            pl.BlockSpec((1, RESO, RESO), lambda b: (b, 0, 0)),
        ],
        out_specs=pl.BlockSpec((1, RESO, RESO), lambda b: (b, 0, 0)),
    )(px, pz)


def _dot(a, b):
    return jax.lax.dot_general(a, b, (((1,), (0,)), ((), ())),
                               preferred_element_type=jnp.float32)


def _block0_body(p_ref, fpW_ref, fpb_ref, W0_ref, b0_ref, W1_ref, b1_ref,
                 Ws_ref, net_ref, sum_ref):
    c = pl.program_id(1)
    x = _dot(p_ref[0], fpW_ref[...]) + fpb_ref[...]
    h = _dot(jax.nn.relu(x), W0_ref[...]) + b0_ref[...]
    dx = _dot(jax.nn.relu(h), W1_ref[...]) + b1_ref[...]
    out = _dot(x, Ws_ref[...]) + dx
    net_ref[0] = out
    part = jnp.sum(out, axis=0, keepdims=True)

    @pl.when(c == 0)
    def _():
        sum_ref[0] = part

    @pl.when(c != 0)
    def _():
        sum_ref[0] = sum_ref[0] + part


def _run_block0(p, fpW, fpb, W0, b0, W1, b1, Ws):
    full = lambda shape: pl.BlockSpec(shape, lambda b, c: tuple(0 for _ in shape))
    return pl.pallas_call(
        _block0_body,
        out_shape=(
            jax.ShapeDtypeStruct((B, T, H), jnp.float32),
            jax.ShapeDtypeStruct((B, 1, H), jnp.float32),
        ),
        grid=(B, NCHUNK),
        in_specs=[
            pl.BlockSpec((1, RC, DIM), lambda b, c: (b, c, 0)),
            full((DIM, 3 * H)), full((1, 3 * H)),
            full((3 * H, H)), full((1, H)),
            full((H, H)), full((1, H)),
            full((3 * H, H)),
        ],
        out_specs=(
            pl.BlockSpec((1, RC, H), lambda b, c: (b, c, 0)),
            pl.BlockSpec((1, 1, H), lambda b, c: (b, 0, 0)),
        ),
    )(p, fpW, fpb, W0, b0, W1, b1, Ws)


def _blocki_body(last, net_in_ref, pool_ref, sum_in_ref,
                 W0np_ref, W0m_ref, b0_ref, W1_ref, b1_ref,
                 Wsnp_ref, Wsm_ref, fcc_ref, fcb_ref, *outs):
    c = pl.program_id(1)
    xn = net_in_ref[0]
    xp = pool_ref[0]
    xm = sum_in_ref[0] * (1.0 / T)
    xcat = jnp.concatenate([xn, xp], axis=1)
    h = (_dot(jax.nn.relu(xcat), W0np_ref[...])
         + _dot(jax.nn.relu(xm), W0m_ref[...])
         + b0_ref[...])
    dx = _dot(jax.nn.relu(h), W1_ref[...]) + b1_ref[...]
    out = (_dot(xcat, Wsnp_ref[...])
           + _dot(xm, Wsm_ref[...]) + dx)
    if last:
        (c_ref,) = outs
        c_ref[0] = _dot(out, fcc_ref[...]) + fcb_ref[...]
    else:
        net_ref, sum_ref = outs
        net_ref[0] = out
        part = jnp.sum(out, axis=0, keepdims=True)

        @pl.when(c == 0)
        def _():
            sum_ref[0] = part

        @pl.when(c != 0)
        def _():
            sum_ref[0] = sum_ref[0] + part


def _run_blocki(net, pooled, sum_in, W0np, W0m, b0, W1, b1,
                Wsnp, Wsm, fcc, fcb, last):
    full = lambda shape: pl.BlockSpec(shape, lambda b, c: tuple(0 for _ in shape))
    if last:
        out_shape = (jax.ShapeDtypeStruct((B, T, H), jnp.float32),)
        out_specs = (pl.BlockSpec((1, RC, H), lambda b, c: (b, c, 0)),)
    else:
        out_shape = (
            jax.ShapeDtypeStruct((B, T, H), jnp.float32),
            jax.ShapeDtypeStruct((B, 1, H), jnp.float32),
        )
        out_specs = (
            pl.BlockSpec((1, RC, H), lambda b, c: (b, c, 0)),
            pl.BlockSpec((1, 1, H), lambda b, c: (b, 0, 0)),
        )
    return pl.pallas_call(
        functools.partial(_blocki_body, last),
        out_shape=out_shape,
        grid=(B, NCHUNK),
        in_specs=[
            pl.BlockSpec((1, RC, H), lambda b, c: (b, c, 0)),
            pl.BlockSpec((1, RC, H), lambda b, c: (b, c, 0)),
            pl.BlockSpec((1, 1, H), lambda b, c: (b, 0, 0)),
            full((2 * H, H)), full((H, H)), full((1, H)),
            full((H, H)), full((1, H)),
            full((2 * H, H)), full((H, H)),
            full((H, H)), full((1, H)),
        ],
        out_specs=out_specs,
    )(net, pooled, sum_in, W0np, W0m, b0, W1, b1, Wsnp, Wsm, fcc, fcb)


def _mesh():
    return plsc.VectorSubcoreMesh(core_axis_name="c", subcore_axis_name="s")


def _worker_id():
    return lax.axis_index("s") * 2 + lax.axis_index("c")


def _sget(ref, i):
    return ref[pl.ds(i, L)][0]


def _select(idx_v, sel_pt, sel_bin, w, own_fn, bin_fn, pad_bin=None):
    lanes = lax.iota(jnp.int32, L)

    def body(t, cnt):
        v = idx_v[pl.ds(t * L, L)]
        own = own_fn(v, w)
        ones = own.astype(jnp.int32)
        pos = cnt + plsc.cumsum(ones) - 1
        plsc.store_scatter(sel_pt, [pos], t * L + lanes, mask=own)
        plsc.store_scatter(sel_bin, [pos], bin_fn(v, w), mask=own)
        return cnt + jnp.sum(ones)

    cnt = lax.fori_loop(0, T // L, body, jnp.int32(0), unroll=4)
    rup = ((cnt + CH - 1) // CH) * CH

    @pl.when(cnt > 0)
    def _():
        lp = jnp.full((L,), _sget(sel_pt, cnt - 1), jnp.int32)
        if pad_bin is None:
            lb = jnp.full((L,), _sget(sel_bin, cnt - 1), jnp.int32)
        else:
            lb = jnp.full((L,), pad_bin, jnp.int32)
        for m in range(0, CH, L):
            sel_pt[pl.ds(cnt + m, L)] = lp
            sel_bin[pl.ds(cnt + m, L)] = lb

    return cnt, rup


def _pool_sc(idx, net):
    @functools.partial(
        pl.kernel,
        mesh=_mesh(),
        compiler_params=pltpu.CompilerParams(needs_layout_passes=False),
        out_type=jax.ShapeDtypeStruct((B, T, H), jnp.float32),
        scratch_types=[
            pltpu.VMEM((T,), jnp.int32),
            pltpu.VMEM((T + 2 * CH,), jnp.int32),
            pltpu.VMEM((T + 2 * CH,), jnp.int32),
            pltpu.VMEM((BPW * H,), jnp.float32),
            pltpu.VMEM((2, CH, H), jnp.float32),
            pltpu.SemaphoreType.DMA,
            pltpu.SemaphoreType.DMA,
            pltpu.SemaphoreType.DMA,
        ],
    )
    def kern(idx_hbm, net_hbm, out_hbm, idx_v, sel_pt, sel_bin, bins, gstage,
             sem_a, sem_b, sem_w):
        w = _worker_id()
        neg = jnp.full((L,), _NEG, jnp.float32)

        def one_batch(b, _):
            pltpu.sync_copy(idx_hbm.at[b], idx_v)
            with jax.named_scope("psel"):
                cnt, rup = _select(idx_v, sel_pt, sel_bin, w,
                                   lambda v, w: (v & (NW - 1)) == w,
                                   lambda v, w: v >> 5)
            nch = rup // CH

            def init_group(k, _):
                def init_pt(j, _):
                    base = _sget(sel_bin, k * CH + j) * H
                    for f in range(H // L):
                        bins[pl.ds(base + f * L, L)] = neg
                    return 0

                return lax.fori_loop(0, CH, init_pt, 0, unroll=4)

            with jax.named_scope("pinit"):
                lax.fori_loop(0, nch, init_group, 0)

            def fire(k, buf, sem):
                pltpu.async_copy(
                    net_hbm.at[b].at[sel_pt.at[pl.ds(k * CH, CH)]],
                    gstage.at[buf], sem)

            def drain_g(sem):
                pltpu.make_async_copy(net_hbm.at[b].at[pl.ds(0, CH)],
                                      gstage.at[0], sem).wait()

            def process(k, buf):
                def one(j, _):
                    base = _sget(sel_bin, k * CH + j) * H
                    for f in range(H // L):
                        s = pl.ds(base + f * L, L)
                        bins[s] = jnp.maximum(
                            bins[s], gstage[buf, j, pl.ds(f * L, L)])
                    return 0

                lax.fori_loop(0, CH, one, 0, unroll=4)

            @pl.when(nch > 0)
            def _():
                fire(0, 0, sem_a)

            def rmw_chunk(k, _):
                @pl.when(lax.rem(k, 2) == 0)
                def _():
                    @pl.when(k + 1 < nch)
                    def _():
                        fire(k + 1, 1, sem_b)
                    drain_g(sem_a)
                    process(k, 0)

                @pl.when(lax.rem(k, 2) == 1)
                def _():
                    @pl.when(k + 1 < nch)
                    def _():
                        fire(k + 1, 0, sem_a)
                    drain_g(sem_b)
                    process(k, 1)
                return 0

            with jax.named_scope("prmw"):
                lax.fori_loop(0, nch, rmw_chunk, 0)

            def gb_group(k, _):
                def fire(j, _):
                    q = k * CH + j
                    base = _sget(sel_bin, q) * H
                    pt = _sget(sel_pt, q)
                    pltpu.async_copy(bins.at[pl.ds(base, H)],
                                     out_hbm.at[b, pt], sem_w)
                    return 0

                lax.fori_loop(0, CH, fire, 0, unroll=4)

                def drain(j, _):
                    pltpu.make_async_copy(out_hbm.at[b, 0],
                                          bins.at[pl.ds(0, H)], sem_w).wait()
                    return 0

                lax.fori_loop(0, CH, drain, 0, unroll=4)
                return 0

            with jax.named_scope("pgb"):
                lax.fori_loop(0, nch, gb_group, 0)
            return 0

        lax.fori_loop(0, B, one_batch, 0)

    return kern(idx, net)


def _scatter_mean_sc(idx, cfeat):
    @functools.partial(
        pl.kernel,
        mesh=_mesh(),
        compiler_params=pltpu.CompilerParams(needs_layout_passes=False),
        out_type=jax.ShapeDtypeStruct((B, C_DIM, RESO, RESO), jnp.float32),
        scratch_types=[
            pltpu.VMEM((T,), jnp.int32),
            pltpu.VMEM((T + 2 * CH,), jnp.int32),
            pltpu.VMEM((T + 2 * CH,), jnp.int32),
            pltpu.VMEM((BPW * (H + 1),), jnp.float32),
            pltpu.VMEM((BPW + L,), jnp.float32),
            pltpu.VMEM((BPW,), jnp.float32),
            pltpu.VMEM((CH, H), jnp.float32),
            pltpu.VMEM((8, BPW), jnp.float32),
            pltpu.SemaphoreType.DMA,
            pltpu.SemaphoreType.DMA,
        ],
    )
    def kern(idx_hbm, c_hbm, out_hbm, idx_v, sel_pt, sel_bin, bins, cnt_v,
             inv_v, gstage, ring, sem, sem_w):
        w = _worker_id()
        zeros = jnp.zeros((L,), jnp.float32)
        lanes = lax.iota(jnp.int32, L)
        NG = BPW // RESO
        SM = H + 1

        def one_batch(b, _):
            def z0(i, _):
                bins[pl.ds(i * L, L)] = zeros
                return 0

            with jax.named_scope("mz0"):
                lax.fori_loop(0, BPW * SM // L, z0, 0, unroll=8)

            def zc(i, _):
                cnt_v[pl.ds(i * L, L)] = zeros
                return 0

            lax.fori_loop(0, BPW // L, zc, 0, unroll=4)

            pltpu.sync_copy(idx_hbm.at[b], idx_v)
            with jax.named_scope("msel"):
                cnt, rup = _select(
                    idx_v, sel_pt, sel_bin, w,
                    lambda v, w: ((v >> 7) & (NW - 1)) == w,
                    lambda v, w: ((v >> 12) << 7) | (v & (RESO - 1)))
            nch = rup // CH

            def rmw_chunk(k, _):
                pltpu.async_copy(
                    c_hbm.at[b].at[sel_pt.at[pl.ds(k * CH, CH)]], gstage,
                    sem).wait()

                def one(j, _):
                    q = k * CH + j

                    @pl.when(q < cnt)
                    def _():
                        bl = _sget(sel_bin, q)
                        blv = jnp.full((L,), bl, jnp.int32)
                        cv = plsc.load_gather(cnt_v, [blv]) + 1.0
                        plsc.store_scatter(cnt_v, [blv], cv)
                        base = bl * SM
                        for f in range(H // L):
                            s = pl.ds(base + f * L, L)
                            bins[s] = bins[s] + gstage[j, pl.ds(f * L, L)]
                    return 0

                return lax.fori_loop(0, CH, one, 0, unroll=4)

            with jax.named_scope("mrmw"):
                lax.fori_loop(0, nch, rmw_chunk, 0)

            def invc(i, _):
                cv = cnt_v[pl.ds(i * L, L)]
                inv_v[pl.ds(i * L, L)] = 1.0 / jnp.maximum(cv, 1.0)
                return 0

            lax.fori_loop(0, BPW // L, invc, 0, unroll=4)

            def orow(f, _):
                r = f & 7

                @pl.when(f >= 8)
                def _():
                    def drain(j, _):
                        pltpu.make_async_copy(out_hbm.at[b, 0, 0],
                                              ring.at[0, pl.ds(0, RESO)],
                                              sem_w).wait()
                        return 0
                    lax.fori_loop(0, NG, drain, 0, unroll=4)

                def gcol(i, _):
                    iv = (i * L + lanes) * SM + f
                    vals = plsc.load_gather(bins, [iv]) * inv_v[pl.ds(i * L, L)]
                    ring[r, pl.ds(i * L, L)] = vals
                    return 0

                lax.fori_loop(0, BPW // L, gcol, 0, unroll=4)

                for g in range(NG):
                    pltpu.async_copy(ring.at[r, pl.ds(g * RESO, RESO)],
                                     out_hbm.at[b, f, w + NW * g], sem_w)
                return 0

            with jax.named_scope("mout"):
                lax.fori_loop(0, H, orow, 0)

                def draintail(j, _):
                    pltpu.make_async_copy(out_hbm.at[b, 0, 0],
                                          ring.at[0, pl.ds(0, RESO)],
                                          sem_w).wait()
                    return 0

                lax.fori_loop(0, 8 * NG, draintail, 0, unroll=4)
            return 0

        lax.fori_loop(0, B, one_batch, 0)

    return kern(idx, cfeat)


def kernel(p, fc_pos_W, fc_pos_b, blocks_fc0_W, blocks_fc0_b, blocks_fc1_W,
           blocks_fc1_b, blocks_sc_W, fc_c_W, fc_c_b):
    px = p[:, :, 0].reshape(B, RESO, RESO)
    pz = p[:, :, 2].reshape(B, RESO, RESO)
    idx3 = _compute_idx(px, pz)
    idx = idx3.reshape(B, T)

    fpb = fc_pos_b.reshape(1, 3 * H)
    net, s = _run_block0(p, fc_pos_W, fpb,
                         blocks_fc0_W[0], blocks_fc0_b[0].reshape(1, H),
                         blocks_fc1_W[0], blocks_fc1_b[0].reshape(1, H),
                         blocks_sc_W[0])

    for i in range(1, N_BLOCKS):
        last = i == N_BLOCKS - 1
        pooled = _pool_sc(idx, net)
        W0 = blocks_fc0_W[i]
        Ws = blocks_sc_W[i]
        outs = _run_blocki(
            net, pooled, s,
            W0[:2 * H], W0[2 * H:], blocks_fc0_b[i].reshape(1, H),
            blocks_fc1_W[i], blocks_fc1_b[i].reshape(1, H),
            Ws[:2 * H], Ws[2 * H:],
            fc_c_W, fc_c_b.reshape(1, H), last)
        if last:
            (cfeat,) = outs
        else:
            net, s = outs

    return _scatter_mean_sc(idx, cfeat)


# speedup vs baseline: 3.9099x; 1.0271x over previous
import functools

import jax
import jax.numpy as jnp
from jax import lax
from jax.experimental import pallas as pl
from jax.experimental.pallas import tpu as pltpu
from jax.experimental.pallas import tpu_sc as plsc

B, T, DIM = 4, 16384, 3
C_DIM, HIDDEN, N_BLOCKS = 128, 128, 5
RESO, PADDING, SCALE = 128, 0.1, 8.0
NSEG = RESO * RESO
H = HIDDEN
RC = 2048
NCHUNK = T // RC

NW = 32
BPW = NSEG // NW
L = 16
CH = 48

_NEG = -3.0e38


def _idx_body(px_ref, pz_ref, o_ref):
    scale = 1.0 / (SCALE * (1.0 + PADDING + 1e-3))
    x = jnp.clip(px_ref[0] * scale + 0.5, 0.0, 1.0 - 1e-3)
    z = jnp.clip(pz_ref[0] * scale + 0.5, 0.0, 1.0 - 1e-3)
    gx = (x * RESO).astype(jnp.int32)
    gz = (z * RESO).astype(jnp.int32)
    o_ref[0] = gx + RESO * gz


def _compute_idx(px, pz):
    return pl.pallas_call(
        _idx_body,
        out_shape=jax.ShapeDtypeStruct((B, RESO, RESO), jnp.int32),
        grid=(B,),
        in_specs=[
            pl.BlockSpec((1, RESO, RESO), lambda b: (b, 0, 0)),
            pl.BlockSpec((1, RESO, RESO), lambda b: (b, 0, 0)),
        ],
        out_specs=pl.BlockSpec((1, RESO, RESO), lambda b: (b, 0, 0)),
    )(px, pz)


def _dot(a, b):
    return jax.lax.dot_general(a, b, (((1,), (0,)), ((), ())),
                               preferred_element_type=jnp.float32)


def _block0_body(p_ref, fpW_ref, fpb_ref, W0_ref, b0_ref, W1_ref, b1_ref,
                 Ws_ref, net_ref, sum_ref):
    c = pl.program_id(1)
    x = _dot(p_ref[0], fpW_ref[...]) + fpb_ref[...]
    h = _dot(jax.nn.relu(x), W0_ref[...]) + b0_ref[...]
    dx = _dot(jax.nn.relu(h), W1_ref[...]) + b1_ref[...]
    out = _dot(x, Ws_ref[...]) + dx
    net_ref[0] = out
    part = jnp.sum(out, axis=0, keepdims=True)

    @pl.when(c == 0)
    def _():
        sum_ref[0] = part

    @pl.when(c != 0)
    def _():
        sum_ref[0] = sum_ref[0] + part


def _run_block0(p, fpW, fpb, W0, b0, W1, b1, Ws):
    full = lambda shape: pl.BlockSpec(shape, lambda b, c: tuple(0 for _ in shape))
    return pl.pallas_call(
        _block0_body,
        out_shape=(
            jax.ShapeDtypeStruct((B, T, H), jnp.float32),
            jax.ShapeDtypeStruct((B, 1, H), jnp.float32),
        ),
        grid=(B, NCHUNK),
        in_specs=[
            pl.BlockSpec((1, RC, DIM), lambda b, c: (b, c, 0)),
            full((DIM, 3 * H)), full((1, 3 * H)),
            full((3 * H, H)), full((1, H)),
            full((H, H)), full((1, H)),
            full((3 * H, H)),
        ],
        out_specs=(
            pl.BlockSpec((1, RC, H), lambda b, c: (b, c, 0)),
            pl.BlockSpec((1, 1, H), lambda b, c: (b, 0, 0)),
        ),
    )(p, fpW, fpb, W0, b0, W1, b1, Ws)


def _blocki_body(last, net_in_ref, pool_ref, sum_in_ref,
                 W0np_ref, W0m_ref, b0_ref, W1_ref, b1_ref,
                 Wsnp_ref, Wsm_ref, fcc_ref, fcb_ref, *outs):
    c = pl.program_id(1)
    xn = net_in_ref[0]
    xp = pool_ref[0]
    xm = sum_in_ref[0] * (1.0 / T)
    xcat = jnp.concatenate([xn, xp], axis=1)
    h = (_dot(jax.nn.relu(xcat), W0np_ref[...])
         + _dot(jax.nn.relu(xm), W0m_ref[...])
         + b0_ref[...])
    dx = _dot(jax.nn.relu(h), W1_ref[...]) + b1_ref[...]
    out = (_dot(xcat, Wsnp_ref[...])
           + _dot(xm, Wsm_ref[...]) + dx)
    if last:
        (c_ref,) = outs
        c_ref[0] = _dot(out, fcc_ref[...]) + fcb_ref[...]
    else:
        net_ref, sum_ref = outs
        net_ref[0] = out
        part = jnp.sum(out, axis=0, keepdims=True)

        @pl.when(c == 0)
        def _():
            sum_ref[0] = part

        @pl.when(c != 0)
        def _():
            sum_ref[0] = sum_ref[0] + part


def _run_blocki(net, pooled, sum_in, W0np, W0m, b0, W1, b1,
                Wsnp, Wsm, fcc, fcb, last):
    full = lambda shape: pl.BlockSpec(shape, lambda b, c: tuple(0 for _ in shape))
    if last:
        out_shape = (jax.ShapeDtypeStruct((B, T, H), jnp.float32),)
        out_specs = (pl.BlockSpec((1, RC, H), lambda b, c: (b, c, 0)),)
    else:
        out_shape = (
            jax.ShapeDtypeStruct((B, T, H), jnp.float32),
            jax.ShapeDtypeStruct((B, 1, H), jnp.float32),
        )
        out_specs = (
            pl.BlockSpec((1, RC, H), lambda b, c: (b, c, 0)),
            pl.BlockSpec((1, 1, H), lambda b, c: (b, 0, 0)),
        )
    return pl.pallas_call(
        functools.partial(_blocki_body, last),
        out_shape=out_shape,
        grid=(B, NCHUNK),
        in_specs=[
            pl.BlockSpec((1, RC, H), lambda b, c: (b, c, 0)),
            pl.BlockSpec((1, RC, H), lambda b, c: (b, c, 0)),
            pl.BlockSpec((1, 1, H), lambda b, c: (b, 0, 0)),
            full((2 * H, H)), full((H, H)), full((1, H)),
            full((H, H)), full((1, H)),
            full((2 * H, H)), full((H, H)),
            full((H, H)), full((1, H)),
        ],
        out_specs=out_specs,
    )(net, pooled, sum_in, W0np, W0m, b0, W1, b1, Wsnp, Wsm, fcc, fcb)


def _mesh():
    return plsc.VectorSubcoreMesh(core_axis_name="c", subcore_axis_name="s")


def _worker_id():
    return lax.axis_index("s") * 2 + lax.axis_index("c")


def _sget(ref, i):
    return ref[pl.ds(i, L)][0]


def _select(idx_v, sel_pt, sel_bin, w, own_fn, bin_fn, pad_bin=None):
    lanes = lax.iota(jnp.int32, L)

    def body(t, cnt):
        v = idx_v[pl.ds(t * L, L)]
        own = own_fn(v, w)
        ones = own.astype(jnp.int32)
        pos = cnt + plsc.cumsum(ones) - 1
        plsc.store_scatter(sel_pt, [pos], t * L + lanes, mask=own)
        plsc.store_scatter(sel_bin, [pos], bin_fn(v, w), mask=own)
        return cnt + jnp.sum(ones)

    cnt = lax.fori_loop(0, T // L, body, jnp.int32(0), unroll=4)
    rup = ((cnt + CH - 1) // CH) * CH

    @pl.when(cnt > 0)
    def _():
        lp = jnp.full((L,), _sget(sel_pt, cnt - 1), jnp.int32)
        if pad_bin is None:
            lb = jnp.full((L,), _sget(sel_bin, cnt - 1), jnp.int32)
        else:
            lb = jnp.full((L,), pad_bin, jnp.int32)
        for m in range(0, CH, L):
            sel_pt[pl.ds(cnt + m, L)] = lp
            sel_bin[pl.ds(cnt + m, L)] = lb

    return cnt, rup


def _pool_sc(idx, net):
    @functools.partial(
        pl.kernel,
        mesh=_mesh(),
        compiler_params=pltpu.CompilerParams(needs_layout_passes=False),
        out_type=jax.ShapeDtypeStruct((B, T, H), jnp.float32),
        scratch_types=[
            pltpu.VMEM((T,), jnp.int32),
            pltpu.VMEM((T + 2 * CH,), jnp.int32),
            pltpu.VMEM((T + 2 * CH,), jnp.int32),
            pltpu.VMEM((BPW * H,), jnp.float32),
            pltpu.VMEM((2, CH, H), jnp.float32),
            pltpu.SemaphoreType.DMA,
            pltpu.SemaphoreType.DMA,
            pltpu.SemaphoreType.DMA,
        ],
    )
    def kern(idx_hbm, net_hbm, out_hbm, idx_v, sel_pt, sel_bin, bins, gstage,
             sem_a, sem_b, sem_w):
        w = _worker_id()
        neg = jnp.full((L,), _NEG, jnp.float32)

        def one_batch(b, _):
            pltpu.sync_copy(idx_hbm.at[b], idx_v)
            with jax.named_scope("psel"):
                cnt, rup = _select(idx_v, sel_pt, sel_bin, w,
                                   lambda v, w: (v & (NW - 1)) == w,
                                   lambda v, w: v >> 5)
            nch = rup // CH

            def init_group(k, _):
                def init_pt(j, _):
                    base = _sget(sel_bin, k * CH + j) * H
                    for f in range(H // L):
                        bins[pl.ds(base + f * L, L)] = neg
                    return 0

                return lax.fori_loop(0, CH, init_pt, 0, unroll=4)

            with jax.named_scope("pinit"):
                lax.fori_loop(0, nch, init_group, 0)

            def fire(k, buf, sem):
                pltpu.async_copy(
                    net_hbm.at[b].at[sel_pt.at[pl.ds(k * CH, CH)]],
                    gstage.at[buf], sem)

            def drain_g(sem):
                pltpu.make_async_copy(net_hbm.at[b].at[pl.ds(0, CH)],
                                      gstage.at[0], sem).wait()

            def process(k, buf):
                def one(j, _):
                    base = _sget(sel_bin, k * CH + j) * H
                    for f in range(H // L):
                        s = pl.ds(base + f * L, L)
                        bins[s] = jnp.maximum(
                            bins[s], gstage[buf, j, pl.ds(f * L, L)])
                    return 0

                lax.fori_loop(0, CH, one, 0, unroll=4)

            @pl.when(nch > 0)
            def _():
                fire(0, 0, sem_a)

            def rmw_chunk(k, _):
                @pl.when(lax.rem(k, 2) == 0)
                def _():
                    @pl.when(k + 1 < nch)
                    def _():
                        fire(k + 1, 1, sem_b)
                    drain_g(sem_a)
                    process(k, 0)

                @pl.when(lax.rem(k, 2) == 1)
                def _():
                    @pl.when(k + 1 < nch)
                    def _():
                        fire(k + 1, 0, sem_a)
                    drain_g(sem_b)
                    process(k, 1)
                return 0

            with jax.named_scope("prmw"):
                lax.fori_loop(0, nch, rmw_chunk, 0)

            def gb_group(k, _):
                def fire(j, _):
                    q = k * CH + j
                    base = _sget(sel_bin, q) * H
                    pt = _sget(sel_pt, q)
                    pltpu.async_copy(bins.at[pl.ds(base, H)],
                                     out_hbm.at[b, pt], sem_w)
                    return 0

                lax.fori_loop(0, CH, fire, 0, unroll=4)

                def drain(j, _):
                    pltpu.make_async_copy(out_hbm.at[b, 0],
                                          bins.at[pl.ds(0, H)], sem_w).wait()
                    return 0

                lax.fori_loop(0, CH, drain, 0, unroll=4)
                return 0

            with jax.named_scope("pgb"):
                lax.fori_loop(0, nch, gb_group, 0)
            return 0

        lax.fori_loop(0, B, one_batch, 0)

    return kern(idx, net)


def _scatter_mean_sc(idx, cfeat):
    @functools.partial(
        pl.kernel,
        mesh=_mesh(),
        compiler_params=pltpu.CompilerParams(needs_layout_passes=False),
        out_type=jax.ShapeDtypeStruct((B, C_DIM, RESO, RESO), jnp.float32),
        scratch_types=[
            pltpu.VMEM((T,), jnp.int32),
            pltpu.VMEM((T + 2 * CH,), jnp.int32),
            pltpu.VMEM((T + 2 * CH,), jnp.int32),
            pltpu.VMEM((BPW * (H + 1),), jnp.float32),
            pltpu.VMEM((BPW + L,), jnp.float32),
            pltpu.VMEM((BPW,), jnp.float32),
            pltpu.VMEM((2, CH, H), jnp.float32),
            pltpu.VMEM((4, BPW), jnp.float32),
            pltpu.SemaphoreType.DMA,
            pltpu.SemaphoreType.DMA,
            pltpu.SemaphoreType.DMA,
        ],
    )
    def kern(idx_hbm, c_hbm, out_hbm, idx_v, sel_pt, sel_bin, bins, cnt_v,
             inv_v, gstage, ring, sem_a, sem_b, sem_w):
        w = _worker_id()
        zeros = jnp.zeros((L,), jnp.float32)
        lanes = lax.iota(jnp.int32, L)
        NG = BPW // RESO
        SM = H + 1

        def one_batch(b, _):
            def z0(i, _):
                bins[pl.ds(i * L, L)] = zeros
                return 0

            with jax.named_scope("mz0"):
                lax.fori_loop(0, BPW * SM // L, z0, 0, unroll=8)

            def zc(i, _):
                cnt_v[pl.ds(i * L, L)] = zeros
                return 0

            lax.fori_loop(0, BPW // L, zc, 0, unroll=4)

            pltpu.sync_copy(idx_hbm.at[b], idx_v)
            with jax.named_scope("msel"):
                cnt, rup = _select(
                    idx_v, sel_pt, sel_bin, w,
                    lambda v, w: ((v >> 7) & (NW - 1)) == w,
                    lambda v, w: ((v >> 12) << 7) | (v & (RESO - 1)))
            nch = rup // CH

            def fire(k, buf, sem):
                pltpu.async_copy(
                    c_hbm.at[b].at[sel_pt.at[pl.ds(k * CH, CH)]],
                    gstage.at[buf], sem)

            def drain_g(sem):
                pltpu.make_async_copy(c_hbm.at[b].at[pl.ds(0, CH)],
                                      gstage.at[0], sem).wait()

            def process(k, buf):
                def one(j, _):
                    q = k * CH + j

                    @pl.when(q < cnt)
                    def _():
                        bl = _sget(sel_bin, q)
                        blv = jnp.full((L,), bl, jnp.int32)
                        cv = plsc.load_gather(cnt_v, [blv]) + 1.0
                        plsc.store_scatter(cnt_v, [blv], cv)
                        base = bl * SM
                        for f in range(H // L):
                            s = pl.ds(base + f * L, L)
                            bins[s] = bins[s] + gstage[buf, j, pl.ds(f * L, L)]
                    return 0

                lax.fori_loop(0, CH, one, 0, unroll=4)

            @pl.when(nch > 0)
            def _():
                fire(0, 0, sem_a)

            def rmw_chunk(k, _):
                @pl.when(lax.rem(k, 2) == 0)
                def _():
                    @pl.when(k + 1 < nch)
                    def _():
                        fire(k + 1, 1, sem_b)
                    drain_g(sem_a)
                    process(k, 0)

                @pl.when(lax.rem(k, 2) == 1)
                def _():
                    @pl.when(k + 1 < nch)
                    def _():
                        fire(k + 1, 0, sem_a)
                    drain_g(sem_b)
                    process(k, 1)
                return 0

            with jax.named_scope("mrmw"):
                lax.fori_loop(0, nch, rmw_chunk, 0)

            def invc(i, _):
                cv = cnt_v[pl.ds(i * L, L)]
                inv_v[pl.ds(i * L, L)] = 1.0 / jnp.maximum(cv, 1.0)
                return 0

            lax.fori_loop(0, BPW // L, invc, 0, unroll=4)

            def orow(f, _):
                r = f & 3

                @pl.when(f >= 4)
                def _():
                    def drain(j, _):
                        pltpu.make_async_copy(out_hbm.at[b, 0, 0],
                                              ring.at[0, pl.ds(0, RESO)],
                                              sem_w).wait()
                        return 0
                    lax.fori_loop(0, NG, drain, 0, unroll=4)

                def gcol(i, _):
                    iv = (i * L + lanes) * SM + f
                    vals = plsc.load_gather(bins, [iv]) * inv_v[pl.ds(i * L, L)]
                    ring[r, pl.ds(i * L, L)] = vals
                    return 0

                lax.fori_loop(0, BPW // L, gcol, 0, unroll=4)

                for g in range(NG):
                    pltpu.async_copy(ring.at[r, pl.ds(g * RESO, RESO)],
                                     out_hbm.at[b, f, w + NW * g], sem_w)
                return 0

            with jax.named_scope("mout"):
                lax.fori_loop(0, H, orow, 0)

                def draintail(j, _):
                    pltpu.make_async_copy(out_hbm.at[b, 0, 0],
                                          ring.at[0, pl.ds(0, RESO)],
                                          sem_w).wait()
                    return 0

                lax.fori_loop(0, 4 * NG, draintail, 0, unroll=4)
            return 0

        lax.fori_loop(0, B, one_batch, 0)

    return kern(idx, cfeat)


def kernel(p, fc_pos_W, fc_pos_b, blocks_fc0_W, blocks_fc0_b, blocks_fc1_W,
           blocks_fc1_b, blocks_sc_W, fc_c_W, fc_c_b):
    px = p[:, :, 0].reshape(B, RESO, RESO)
    pz = p[:, :, 2].reshape(B, RESO, RESO)
    idx3 = _compute_idx(px, pz)
    idx = idx3.reshape(B, T)

    fpb = fc_pos_b.reshape(1, 3 * H)
    net, s = _run_block0(p, fc_pos_W, fpb,
                         blocks_fc0_W[0], blocks_fc0_b[0].reshape(1, H),
                         blocks_fc1_W[0], blocks_fc1_b[0].reshape(1, H),
                         blocks_sc_W[0])

    for i in range(1, N_BLOCKS):
        last = i == N_BLOCKS - 1
        pooled = _pool_sc(idx, net)
        W0 = blocks_fc0_W[i]
        Ws = blocks_sc_W[i]
        outs = _run_blocki(
            net, pooled, s,
            W0[:2 * H], W0[2 * H:], blocks_fc0_b[i].reshape(1, H),
            blocks_fc1_W[i], blocks_fc1_b[i].reshape(1, H),
            Ws[:2 * H], Ws[2 * H:],
            fc_c_W, fc_c_b.reshape(1, H), last)
        if last:
            (cfeat,) = outs
        else:
            net, s = outs

    return _scatter_mean_sc(idx, cfeat)
